# Initial kernel scaffold; baseline (speedup 1.0000x reference)
import jax
import jax.numpy as jnp
from jax.experimental import pallas as pl


def kernel(z, atom_types, edge_index, Wlp, blp, Wap, bap, init_coords, eW1, eb1, eW2, eb2, nW1, nb1, nW2, nb2, cW1, cb1, cW2, gamma, beta, hW1, hb1, hW2, hb2):
    raise NotImplementedError("write your pallas kernel here")


# speedup vs baseline: 2.5467x; 2.5467x over previous
import functools

import jax
import jax.numpy as jnp
from jax import lax
from jax.experimental import pallas as pl
from jax.experimental.pallas import tpu as pltpu
from jax.experimental.pallas import tpu_sc as plsc

B, A, H, L, F, E = 200, 50, 128, 128, 10, 320000
N = B * A
NL = 2

NC, NS = 2, 16
NW = NC * NS
CH = 128
CHUNKS = -(-E // CH)
CHUNKS_PAD = -(-CHUNKS // NW) * NW
KPW = CHUNKS_PAD // NW
KPT = CHUNKS_PAD // NS
EP = CHUNKS_PAD * CH
NPAD = 10240
ROWS_PER_TILE = NPAD // NS
PAD_NODE = N + 16

BE = 2528
GRID_E = EP // BE


def _precompute_body(zr_ref, at_ref, wlp_ref, blp_ref, wap_ref, bap_ref,
                     w1a_ref, w1b_ref, eb1_ref,
                     nf_ref, p_ref, q_ref):
    nf = (jnp.dot(zr_ref[...], wlp_ref[...], preferred_element_type=jnp.float32)
          + blp_ref[...]
          + jnp.dot(at_ref[...], wap_ref[...], preferred_element_type=jnp.float32)
          + bap_ref[...])
    nf_ref[...] = nf
    p_ref[...] = jnp.dot(nf, w1a_ref[...], preferred_element_type=jnp.float32) + eb1_ref[...]
    q_ref[...] = jnp.dot(nf, w1b_ref[...], preferred_element_type=jnp.float32)


def _precompute(zr, atp, wlp, blp, wapp, bap, w1a, w1b, eb1):
    return pl.pallas_call(
        _precompute_body,
        out_shape=[jax.ShapeDtypeStruct((N, H), jnp.float32)] * 3,
    )(zr, atp, wlp, blp, wapp, bap, w1a, w1b, eb1)


def _edge_body(gp_ref, gq_ref, rel_ref, wd_ref, ew2_ref, eb2_ref,
               cw1_ref, cb1_ref, cw2_ref, sel_ref, lane3_ref,
               m_ref, s_ref):
    rel = rel_ref[...]
    dist = rel[:, 3:4]
    g = gp_ref[...] + gq_ref[...] + dist * wd_ref[...]
    m1 = jax.nn.silu(g)
    m = jax.nn.silu(
        jnp.dot(m1, ew2_ref[...], preferred_element_type=jnp.float32)
        + eb2_ref[...])
    cwp = jax.nn.silu(
        jnp.dot(m, cw1_ref[...], preferred_element_type=jnp.float32)
        + cb1_ref[...])
    cw = jnp.clip(jnp.sum(cwp * cw2_ref[...], axis=-1, keepdims=True), -1.0, 1.0)
    m_ref[...] = m
    s_ref[...] = cw * jnp.dot(rel, sel_ref[...],
                              preferred_element_type=jnp.float32) + lane3_ref[...]


def _edge_mlp(gp, gq, rel4, wd, ew2, eb2, cw1, cb1, cw2r, sel, lane3):
    full128 = pl.BlockSpec((H, H), lambda i: (0, 0))
    row128 = pl.BlockSpec((1, H), lambda i: (0, 0))
    return pl.pallas_call(
        _edge_body,
        grid=(GRID_E,),
        in_specs=[
            pl.BlockSpec((BE, H), lambda i: (i, 0)),
            pl.BlockSpec((BE, H), lambda i: (i, 0)),
            pl.BlockSpec((BE, 4), lambda i: (i, 0)),
            row128, full128, row128, full128, row128, row128,
            pl.BlockSpec((4, H), lambda i: (0, 0)), row128,
        ],
        out_specs=[
            pl.BlockSpec((BE, H), lambda i: (i, 0)),
            pl.BlockSpec((BE, H), lambda i: (i, 0)),
        ],
        out_shape=[
            jax.ShapeDtypeStruct((EP, H), jnp.float32),
            jax.ShapeDtypeStruct((EP, H), jnp.float32),
        ],
    )(gp, gq, rel4, wd, ew2, eb2, cw1, cb1, cw2r, sel, lane3)


def _layernorm(xn, gamma, beta):
    mu = jnp.mean(xn, axis=-1, keepdims=True)
    var = jnp.mean((xn - mu) ** 2, axis=-1, keepdims=True)
    return (xn - mu) * jax.lax.rsqrt(var + 1e-5) * gamma + beta


def _node_mid_body(nf_ref, agg_ref, cu_ref, c_ref,
                   nw1a_ref, nw1b_ref, nb1_ref, nw2_ref, nb2_ref,
                   gam_ref, bet_ref, w1a_ref, w1b_ref, eb1_ref,
                   nf2_ref, c2_ref, p_ref, q_ref):
    nf = nf_ref[...]
    agg = agg_ref[...]
    xn = jax.nn.silu(
        jnp.dot(nf, nw1a_ref[...], preferred_element_type=jnp.float32)
        + jnp.dot(agg, nw1b_ref[...], preferred_element_type=jnp.float32)
        + nb1_ref[...])
    xn = jnp.dot(xn, nw2_ref[...], preferred_element_type=jnp.float32) + nb2_ref[...]
    xn = _layernorm(xn, gam_ref[...], bet_ref[...])
    nf2_ref[...] = xn
    cu = cu_ref[...]
    c2_ref[...] = c_ref[...] + cu / (cu[:, 3:4] + 1e-6)
    p_ref[...] = jnp.dot(xn, w1a_ref[...], preferred_element_type=jnp.float32) + eb1_ref[...]
    q_ref[...] = jnp.dot(xn, w1b_ref[...], preferred_element_type=jnp.float32)


def _node_mid(nf, agg, cu16, c16, nw1a, nw1b, nb1, nw2, nb2,
              gam, bet, w1a, w1b, eb1):
    return pl.pallas_call(
        _node_mid_body,
        out_shape=[
            jax.ShapeDtypeStruct((N, H), jnp.float32),
            jax.ShapeDtypeStruct((N, 16), jnp.float32),
            jax.ShapeDtypeStruct((N, H), jnp.float32),
            jax.ShapeDtypeStruct((N, H), jnp.float32),
        ],
    )(nf, agg, cu16, c16, nw1a, nw1b, nb1, nw2, nb2, gam, bet, w1a, w1b, eb1)


def _node_final_body(nf_ref, agg_ref, cu_ref, c_ref,
                     nw1a_ref, nw1b_ref, nb1_ref, nw2_ref, nb2_ref,
                     gam_ref, bet_ref, hw1_ref, hb1_ref, hw2_ref, hb2_ref,
                     out_ref):
    nf = nf_ref[...]
    agg = agg_ref[...]
    xn = jax.nn.silu(
        jnp.dot(nf, nw1a_ref[...], preferred_element_type=jnp.float32)
        + jnp.dot(agg, nw1b_ref[...], preferred_element_type=jnp.float32)
        + nb1_ref[...])
    xn = jnp.dot(xn, nw2_ref[...], preferred_element_type=jnp.float32) + nb2_ref[...]
    xn = _layernorm(xn, gam_ref[...], bet_ref[...])
    cu = cu_ref[...]
    c2 = c_ref[...] + cu / (cu[:, 3:4] + 1e-6)
    hid = jax.nn.silu(
        jnp.dot(xn, hw1_ref[...], preferred_element_type=jnp.float32)
        + hb1_ref[...])
    cd = jnp.dot(hid, hw2_ref[...], preferred_element_type=jnp.float32) + hb2_ref[...]
    out_ref[...] = c2 + cd


def _node_final(nf, agg, cu16, c16, nw1a, nw1b, nb1, nw2, nb2,
                gam, bet, hw1, hb1, hw2p, hb2p):
    return pl.pallas_call(
        _node_final_body,
        out_shape=jax.ShapeDtypeStruct((N, 16), jnp.float32),
    )(nf, agg, cu16, c16, nw1a, nw1b, nb1, nw2, nb2, gam, bet,
      hw1, hb1, hw2p, hb2p)


@functools.cache
def _mesh():
    return plsc.VectorSubcoreMesh(core_axis_name="c", subcore_axis_name="s",
                                  num_cores=NC, num_subcores=NS)


def _sc_gather_body(p_hbm, q_hbm, cx_hbm, cy_hbm, cz_hbm, ridx_hbm, cidx_hbm,
                    gp_hbm, gq_hbm, relt_hbm,
                    rv, cv, gpb, gqb, rxb, ryb, rzb, db, cxv, cyv, czv, sem):
    wid = lax.axis_index("s") * NC + lax.axis_index("c")
    pltpu.sync_copy(cx_hbm, cxv)
    pltpu.sync_copy(cy_hbm, cyv)
    pltpu.sync_copy(cz_hbm, czv)

    def body(k, _):
        j = wid * KPW + k
        pltpu.sync_copy(ridx_hbm.at[j], rv)
        pltpu.sync_copy(cidx_hbm.at[j], cv)
        pltpu.async_copy(p_hbm.at[rv], gpb, sem).wait()
        pltpu.async_copy(q_hbm.at[cv], gqb, sem).wait()
        for v in range(CH // 16):
            sl = pl.ds(v * 16, 16)
            ir = rv[sl]
            ic = cv[sl]
            rx = plsc.load_gather(cxv, [ir]) - plsc.load_gather(cxv, [ic])
            ry = plsc.load_gather(cyv, [ir]) - plsc.load_gather(cyv, [ic])
            rz = plsc.load_gather(czv, [ir]) - plsc.load_gather(czv, [ic])
            d = jnp.clip(rx * rx + ry * ry + rz * rz, 1e-6, 1e6)
            rxb[sl] = rx
            ryb[sl] = ry
            rzb[sl] = rz
            db[sl] = d
        base = j * CH
        pltpu.sync_copy(gpb, gp_hbm.at[pl.ds(base, CH)])
        pltpu.sync_copy(gqb, gq_hbm.at[pl.ds(base, CH)])
        pltpu.sync_copy(rxb, relt_hbm.at[0, pl.ds(base, CH)])
        pltpu.sync_copy(ryb, relt_hbm.at[1, pl.ds(base, CH)])
        pltpu.sync_copy(rzb, relt_hbm.at[2, pl.ds(base, CH)])
        pltpu.sync_copy(db, relt_hbm.at[3, pl.ds(base, CH)])
        return _

    lax.fori_loop(0, KPW, body, None)


@functools.cache
def _sc_gather():
    return pl.kernel(
        _sc_gather_body,
        out_type=[
            jax.ShapeDtypeStruct((EP, H), jnp.float32),
            jax.ShapeDtypeStruct((EP, H), jnp.float32),
            jax.ShapeDtypeStruct((4, EP), jnp.float32),
        ],
        mesh=_mesh(),
        scratch_types=[
            pltpu.VMEM((CH,), jnp.int32),
            pltpu.VMEM((CH,), jnp.int32),
            pltpu.VMEM((CH, H), jnp.float32),
            pltpu.VMEM((CH, H), jnp.float32),
            pltpu.VMEM((CH,), jnp.float32),
            pltpu.VMEM((CH,), jnp.float32),
            pltpu.VMEM((CH,), jnp.float32),
            pltpu.VMEM((CH,), jnp.float32),
            pltpu.VMEM((NPAD,), jnp.float32),
            pltpu.VMEM((NPAD,), jnp.float32),
            pltpu.VMEM((NPAD,), jnp.float32),
            pltpu.SemaphoreType.DMA,
        ],
        compiler_params=pltpu.CompilerParams(needs_layout_passes=False),
    )


def _sc_scatter_body(m_hbm, s_hbm, ridx_hbm, zero_hbm,
                     agg_hbm, cu_hbm,
                     rv, mb, acc_sh):
    cid = lax.axis_index("c")
    sid = lax.axis_index("s")
    rows = pl.ds(sid * ROWS_PER_TILE, ROWS_PER_TILE)
    pltpu.sync_copy(zero_hbm.at[rows], acc_sh.at[rows])
    plsc.subcore_barrier()

    @pl.when(cid == 0)
    def _():
        def body(k, carry):
            j = sid * KPT + k
            pltpu.sync_copy(ridx_hbm.at[j], rv)
            pltpu.sync_copy(m_hbm.at[pl.ds(j * CH, CH)], mb)
            pltpu.sync_copy(mb, acc_sh.at[rv], add=True)
            return carry
        lax.fori_loop(0, KPT, body, None)

    @pl.when(cid == 1)
    def _():
        def body(k, carry):
            j = sid * KPT + k
            pltpu.sync_copy(ridx_hbm.at[j], rv)
            pltpu.sync_copy(s_hbm.at[pl.ds(j * CH, CH)], mb)
            pltpu.sync_copy(mb, acc_sh.at[rv], add=True)
            return carry
        lax.fori_loop(0, KPT, body, None)

    plsc.subcore_barrier()

    @pl.when(cid == 0)
    def _():
        pltpu.sync_copy(acc_sh.at[rows], agg_hbm.at[rows])

    @pl.when(cid == 1)
    def _():
        pltpu.sync_copy(acc_sh.at[rows], cu_hbm.at[rows])


@functools.cache
def _sc_scatter():
    return pl.kernel(
        _sc_scatter_body,
        out_type=[
            jax.ShapeDtypeStruct((NPAD, H), jnp.float32),
            jax.ShapeDtypeStruct((NPAD, H), jnp.float32),
        ],
        mesh=_mesh(),
        scratch_types=[
            pltpu.VMEM((CH,), jnp.int32),
            pltpu.VMEM((CH, H), jnp.float32),
            pltpu.VMEM_SHARED((NPAD, H), jnp.float32),
        ],
    )


def kernel(z, atom_types, edge_index, Wlp, blp, Wap, bap, init_coords,
           eW1, eb1, eW2, eb2, nW1, nb1, nW2, nb2, cW1, cb1, cW2,
           gamma, beta, hW1, hb1, hW2, hb2):
    f32 = jnp.float32
    row = edge_index[0].astype(jnp.int32)
    col = edge_index[1].astype(jnp.int32)
    ridx = jnp.concatenate(
        [row, jnp.full((EP - E,), PAD_NODE, jnp.int32)]).reshape(CHUNKS_PAD, CH)
    cidx = jnp.concatenate(
        [col, jnp.zeros((EP - E,), jnp.int32)]).reshape(CHUNKS_PAD, CH)
    ridx_g = jnp.where(ridx >= N, 0, ridx)

    zr = jnp.broadcast_to(z[:, None, :], (B, A, L)).reshape(N, L)
    atp = jnp.pad(atom_types.astype(f32), ((0, 0), (0, 6)))
    wapp = jnp.pad(Wap.astype(f32), ((0, 6), (0, 0)))
    c16 = jnp.pad(
        jnp.broadcast_to(init_coords[None], (B, A, 3)).reshape(N, 3),
        ((0, 0), (0, 13)))

    blp2 = blp.reshape(1, H)
    bap2 = bap.reshape(1, H)
    sel = jnp.zeros((4, H), f32).at[0, 0].set(1.0).at[1, 1].set(1.0).at[2, 2].set(1.0)
    lane3 = jnp.zeros((1, H), f32).at[0, 3].set(1.0)
    zeros_big = jnp.zeros((NPAD, H), f32)
    hw2p = jnp.pad(hW2, ((0, 0), (0, 13)))
    hb2p = jnp.pad(hb2.reshape(1, 3), ((0, 0), (0, 13)))

    nf, P, Q = _precompute(zr, atp, Wlp, blp2, wapp, bap2,
                           eW1[0, :H], eW1[0, H:2 * H], eb1[0].reshape(1, H))

    for i in range(NL):
        c16p = jnp.pad(c16, ((0, NPAD - N), (0, 0)))
        cx, cy, cz = c16p[:, 0], c16p[:, 1], c16p[:, 2]
        gp, gq, relt = _sc_gather()(P, Q, cx, cy, cz, ridx_g, cidx)
        rel4 = relt.T
        m, s128 = _edge_mlp(gp, gq, rel4,
                            eW1[i, 2 * H].reshape(1, H), eW2[i],
                            eb2[i].reshape(1, H), cW1[i],
                            cb1[i].reshape(1, H), cW2[i].reshape(1, H),
                            sel, lane3)
        agg, cu128 = _sc_scatter()(m, s128, ridx, zeros_big)
        aggN = agg[:N]
        cu16 = cu128[:N, :16]
        if i < NL - 1:
            nf, c16, P, Q = _node_mid(
                nf, aggN, cu16, c16,
                nW1[i, :H], nW1[i, H:], nb1[i].reshape(1, H),
                nW2[i], nb2[i].reshape(1, H),
                gamma[i].reshape(1, H), beta[i].reshape(1, H),
                eW1[i + 1, :H], eW1[i + 1, H:2 * H], eb1[i + 1].reshape(1, H))
        else:
            out16 = _node_final(
                nf, aggN, cu16, c16,
                nW1[i, :H], nW1[i, H:], nb1[i].reshape(1, H),
                nW2[i], nb2[i].reshape(1, H),
                gamma[i].reshape(1, H), beta[i].reshape(1, H),
                hW1, hb1.reshape(1, H // 2), hw2p, hb2p)
    return out16[:, :3]


# speedup vs baseline: 2.8698x; 1.1269x over previous
import functools

import jax
import jax.numpy as jnp
from jax import lax
from jax.experimental import pallas as pl
from jax.experimental.pallas import tpu as pltpu
from jax.experimental.pallas import tpu_sc as plsc

B, A, H, L, F, E = 200, 50, 128, 128, 10, 320000
N = B * A
NL = 2

NC, NS = 2, 16
NW = NC * NS
CH = 128
CHUNKS = -(-E // CH)
KPW = 2 * (-(-(-(-CHUNKS // NW)) // 2))
CHUNKS_PAD = NW * KPW
KPT = CHUNKS_PAD // NS
EP = CHUNKS_PAD * CH
NPAD = 10240
ROWS_PER_TILE = NPAD // NS
PAD_NODE = N + 16

BE = 2560
GRID_E = EP // BE


def _precompute_body(zr_ref, at_ref, wlp_ref, blp_ref, wap_ref, bap_ref,
                     w1a_ref, w1b_ref, eb1_ref,
                     nf_ref, p_ref, q_ref):
    nf = (jnp.dot(zr_ref[...], wlp_ref[...], preferred_element_type=jnp.float32)
          + blp_ref[...]
          + jnp.dot(at_ref[...], wap_ref[...], preferred_element_type=jnp.float32)
          + bap_ref[...])
    nf_ref[...] = nf
    p_ref[...] = jnp.dot(nf, w1a_ref[...], preferred_element_type=jnp.float32) + eb1_ref[...]
    q_ref[...] = jnp.dot(nf, w1b_ref[...], preferred_element_type=jnp.float32)


def _precompute(zr, atp, wlp, blp, wapp, bap, w1a, w1b, eb1):
    return pl.pallas_call(
        _precompute_body,
        out_shape=[jax.ShapeDtypeStruct((N, H), jnp.float32)] * 3,
    )(zr, atp, wlp, blp, wapp, bap, w1a, w1b, eb1)


def _edge_body(gp_ref, gq_ref, rel_ref, wd_ref, ew2_ref, eb2_ref,
               cw1_ref, cb1_ref, cw2_ref, sel_ref, lane3_ref,
               m_ref, s_ref):
    rel = rel_ref[...]
    dist = rel[:, 3:4]
    g = gp_ref[...] + gq_ref[...] + dist * wd_ref[...]
    m1 = jax.nn.silu(g)
    m = jax.nn.silu(
        jnp.dot(m1, ew2_ref[...], preferred_element_type=jnp.float32)
        + eb2_ref[...])
    cwp = jax.nn.silu(
        jnp.dot(m, cw1_ref[...], preferred_element_type=jnp.float32)
        + cb1_ref[...])
    cw = jnp.clip(jnp.sum(cwp * cw2_ref[...], axis=-1, keepdims=True), -1.0, 1.0)
    m_ref[...] = m
    s_ref[...] = cw * jnp.dot(rel, sel_ref[...],
                              preferred_element_type=jnp.float32) + lane3_ref[...]


def _edge_mlp(gp, gq, rel4, wd, ew2, eb2, cw1, cb1, cw2r, sel, lane3):
    full128 = pl.BlockSpec((H, H), lambda i: (0, 0))
    row128 = pl.BlockSpec((1, H), lambda i: (0, 0))
    return pl.pallas_call(
        _edge_body,
        grid=(GRID_E,),
        in_specs=[
            pl.BlockSpec((BE, H), lambda i: (i, 0)),
            pl.BlockSpec((BE, H), lambda i: (i, 0)),
            pl.BlockSpec((BE, 4), lambda i: (i, 0)),
            row128, full128, row128, full128, row128, row128,
            pl.BlockSpec((4, H), lambda i: (0, 0)), row128,
        ],
        out_specs=[
            pl.BlockSpec((BE, H), lambda i: (i, 0)),
            pl.BlockSpec((BE, H), lambda i: (i, 0)),
        ],
        out_shape=[
            jax.ShapeDtypeStruct((EP, H), jnp.float32),
            jax.ShapeDtypeStruct((EP, H), jnp.float32),
        ],
    )(gp, gq, rel4, wd, ew2, eb2, cw1, cb1, cw2r, sel, lane3)


def _layernorm(xn, gamma, beta):
    mu = jnp.mean(xn, axis=-1, keepdims=True)
    var = jnp.mean((xn - mu) ** 2, axis=-1, keepdims=True)
    return (xn - mu) * jax.lax.rsqrt(var + 1e-5) * gamma + beta


def _node_mid_body(nf_ref, agg_ref, cu_ref, c_ref,
                   nw1a_ref, nw1b_ref, nb1_ref, nw2_ref, nb2_ref,
                   gam_ref, bet_ref, w1a_ref, w1b_ref, eb1_ref,
                   nf2_ref, c2_ref, p_ref, q_ref):
    nf = nf_ref[...]
    agg = agg_ref[...]
    xn = jax.nn.silu(
        jnp.dot(nf, nw1a_ref[...], preferred_element_type=jnp.float32)
        + jnp.dot(agg, nw1b_ref[...], preferred_element_type=jnp.float32)
        + nb1_ref[...])
    xn = jnp.dot(xn, nw2_ref[...], preferred_element_type=jnp.float32) + nb2_ref[...]
    xn = _layernorm(xn, gam_ref[...], bet_ref[...])
    nf2_ref[...] = xn
    cu = cu_ref[...]
    c2_ref[...] = c_ref[...] + cu / (cu[:, 3:4] + 1e-6)
    p_ref[...] = jnp.dot(xn, w1a_ref[...], preferred_element_type=jnp.float32) + eb1_ref[...]
    q_ref[...] = jnp.dot(xn, w1b_ref[...], preferred_element_type=jnp.float32)


def _node_mid(nf, agg, cu16, c16, nw1a, nw1b, nb1, nw2, nb2,
              gam, bet, w1a, w1b, eb1):
    return pl.pallas_call(
        _node_mid_body,
        out_shape=[
            jax.ShapeDtypeStruct((N, H), jnp.float32),
            jax.ShapeDtypeStruct((N, 16), jnp.float32),
            jax.ShapeDtypeStruct((N, H), jnp.float32),
            jax.ShapeDtypeStruct((N, H), jnp.float32),
        ],
    )(nf, agg, cu16, c16, nw1a, nw1b, nb1, nw2, nb2, gam, bet, w1a, w1b, eb1)


def _node_final_body(nf_ref, agg_ref, cu_ref, c_ref,
                     nw1a_ref, nw1b_ref, nb1_ref, nw2_ref, nb2_ref,
                     gam_ref, bet_ref, hw1_ref, hb1_ref, hw2_ref, hb2_ref,
                     out_ref):
    nf = nf_ref[...]
    agg = agg_ref[...]
    xn = jax.nn.silu(
        jnp.dot(nf, nw1a_ref[...], preferred_element_type=jnp.float32)
        + jnp.dot(agg, nw1b_ref[...], preferred_element_type=jnp.float32)
        + nb1_ref[...])
    xn = jnp.dot(xn, nw2_ref[...], preferred_element_type=jnp.float32) + nb2_ref[...]
    xn = _layernorm(xn, gam_ref[...], bet_ref[...])
    cu = cu_ref[...]
    c2 = c_ref[...] + cu / (cu[:, 3:4] + 1e-6)
    hid = jax.nn.silu(
        jnp.dot(xn, hw1_ref[...], preferred_element_type=jnp.float32)
        + hb1_ref[...])
    cd = jnp.dot(hid, hw2_ref[...], preferred_element_type=jnp.float32) + hb2_ref[...]
    out_ref[...] = c2 + cd


def _node_final(nf, agg, cu16, c16, nw1a, nw1b, nb1, nw2, nb2,
                gam, bet, hw1, hb1, hw2p, hb2p):
    return pl.pallas_call(
        _node_final_body,
        out_shape=jax.ShapeDtypeStruct((N, 16), jnp.float32),
    )(nf, agg, cu16, c16, nw1a, nw1b, nb1, nw2, nb2, gam, bet,
      hw1, hb1, hw2p, hb2p)


@functools.cache
def _mesh():
    return plsc.VectorSubcoreMesh(core_axis_name="c", subcore_axis_name="s",
                                  num_cores=NC, num_subcores=NS)


def _sc_gather_body(p_hbm, q_hbm, cx_hbm, cy_hbm, cz_hbm, ridx_hbm, cidx_hbm,
                    gp_hbm, gq_hbm, relt_hbm,
                    rv0, cv0, rv1, cv1, gpb0, gqb0, gpb1, gqb1,
                    rxb, ryb, rzb, db, cxv, cyv, czv,
                    sa0, sb0, sa1, sb1):
    wid = lax.axis_index("s") * NC + lax.axis_index("c")
    base = wid * KPW
    pltpu.sync_copy(cx_hbm, cxv)
    pltpu.sync_copy(cy_hbm, cyv)
    pltpu.sync_copy(cz_hbm, czv)

    def fire(j, rv, cv, gpb, gqb, sa, sb):
        pltpu.sync_copy(ridx_hbm.at[j], rv)
        pltpu.sync_copy(cidx_hbm.at[j], cv)
        pltpu.async_copy(p_hbm.at[rv], gpb, sa)
        pltpu.async_copy(q_hbm.at[cv], gqb, sb)

    def process(j, rv, cv, gpb, gqb, sa, sb):
        pltpu.make_async_copy(p_hbm.at[rv], gpb, sa).wait()
        pltpu.make_async_copy(q_hbm.at[cv], gqb, sb).wait()
        for v in range(CH // 16):
            sl = pl.ds(v * 16, 16)
            ir = rv[sl]
            ic = cv[sl]
            rx = plsc.load_gather(cxv, [ir]) - plsc.load_gather(cxv, [ic])
            ry = plsc.load_gather(cyv, [ir]) - plsc.load_gather(cyv, [ic])
            rz = plsc.load_gather(czv, [ir]) - plsc.load_gather(czv, [ic])
            d = jnp.clip(rx * rx + ry * ry + rz * rz, 1e-6, 1e6)
            rxb[sl] = rx
            ryb[sl] = ry
            rzb[sl] = rz
            db[sl] = d
        eb = j * CH
        pltpu.sync_copy(gpb, gp_hbm.at[pl.ds(eb, CH)])
        pltpu.sync_copy(gqb, gq_hbm.at[pl.ds(eb, CH)])
        pltpu.sync_copy(rxb, relt_hbm.at[0, pl.ds(eb, CH)])
        pltpu.sync_copy(ryb, relt_hbm.at[1, pl.ds(eb, CH)])
        pltpu.sync_copy(rzb, relt_hbm.at[2, pl.ds(eb, CH)])
        pltpu.sync_copy(db, relt_hbm.at[3, pl.ds(eb, CH)])

    fire(base, rv0, cv0, gpb0, gqb0, sa0, sb0)
    KH = KPW // 2

    def body(k, _):
        j0 = base + 2 * k
        fire(j0 + 1, rv1, cv1, gpb1, gqb1, sa1, sb1)
        process(j0, rv0, cv0, gpb0, gqb0, sa0, sb0)

        @pl.when(k < KH - 1)
        def _f():
            fire(j0 + 2, rv0, cv0, gpb0, gqb0, sa0, sb0)

        process(j0 + 1, rv1, cv1, gpb1, gqb1, sa1, sb1)
        return _

    lax.fori_loop(0, KH, body, None)


@functools.cache
def _sc_gather():
    return pl.kernel(
        _sc_gather_body,
        out_type=[
            jax.ShapeDtypeStruct((EP, H), jnp.float32),
            jax.ShapeDtypeStruct((EP, H), jnp.float32),
            jax.ShapeDtypeStruct((4, EP), jnp.float32),
        ],
        mesh=_mesh(),
        scratch_types=[
            pltpu.VMEM((CH,), jnp.int32),
            pltpu.VMEM((CH,), jnp.int32),
            pltpu.VMEM((CH,), jnp.int32),
            pltpu.VMEM((CH,), jnp.int32),
            pltpu.VMEM((CH, H), jnp.float32),
            pltpu.VMEM((CH, H), jnp.float32),
            pltpu.VMEM((CH, H), jnp.float32),
            pltpu.VMEM((CH, H), jnp.float32),
            pltpu.VMEM((CH,), jnp.float32),
            pltpu.VMEM((CH,), jnp.float32),
            pltpu.VMEM((CH,), jnp.float32),
            pltpu.VMEM((CH,), jnp.float32),
            pltpu.VMEM((NPAD,), jnp.float32),
            pltpu.VMEM((NPAD,), jnp.float32),
            pltpu.VMEM((NPAD,), jnp.float32),
            pltpu.SemaphoreType.DMA,
            pltpu.SemaphoreType.DMA,
            pltpu.SemaphoreType.DMA,
            pltpu.SemaphoreType.DMA,
        ],
        compiler_params=pltpu.CompilerParams(needs_layout_passes=False),
    )


def _sc_scatter_body(m_hbm, s_hbm, ridx_hbm, zero_hbm,
                     agg_hbm, cu_hbm,
                     rv, mb, acc_sh):
    cid = lax.axis_index("c")
    sid = lax.axis_index("s")
    rows = pl.ds(sid * ROWS_PER_TILE, ROWS_PER_TILE)
    pltpu.sync_copy(zero_hbm.at[rows], acc_sh.at[rows])
    plsc.subcore_barrier()

    @pl.when(cid == 0)
    def _():
        def body(k, carry):
            j = sid * KPT + k
            pltpu.sync_copy(ridx_hbm.at[j], rv)
            pltpu.sync_copy(m_hbm.at[pl.ds(j * CH, CH)], mb)
            pltpu.sync_copy(mb, acc_sh.at[rv], add=True)
            return carry
        lax.fori_loop(0, KPT, body, None)

    @pl.when(cid == 1)
    def _():
        def body(k, carry):
            j = sid * KPT + k
            pltpu.sync_copy(ridx_hbm.at[j], rv)
            pltpu.sync_copy(s_hbm.at[pl.ds(j * CH, CH)], mb)
            pltpu.sync_copy(mb, acc_sh.at[rv], add=True)
            return carry
        lax.fori_loop(0, KPT, body, None)

    plsc.subcore_barrier()

    @pl.when(cid == 0)
    def _():
        pltpu.sync_copy(acc_sh.at[rows], agg_hbm.at[rows])

    @pl.when(cid == 1)
    def _():
        pltpu.sync_copy(acc_sh.at[rows], cu_hbm.at[rows])


@functools.cache
def _sc_scatter():
    return pl.kernel(
        _sc_scatter_body,
        out_type=[
            jax.ShapeDtypeStruct((NPAD, H), jnp.float32),
            jax.ShapeDtypeStruct((NPAD, H), jnp.float32),
        ],
        mesh=_mesh(),
        scratch_types=[
            pltpu.VMEM((CH,), jnp.int32),
            pltpu.VMEM((CH, H), jnp.float32),
            pltpu.VMEM_SHARED((NPAD, H), jnp.float32),
        ],
    )


def kernel(z, atom_types, edge_index, Wlp, blp, Wap, bap, init_coords,
           eW1, eb1, eW2, eb2, nW1, nb1, nW2, nb2, cW1, cb1, cW2,
           gamma, beta, hW1, hb1, hW2, hb2):
    f32 = jnp.float32
    row = edge_index[0].astype(jnp.int32)
    col = edge_index[1].astype(jnp.int32)
    ridx = jnp.concatenate(
        [row, jnp.full((EP - E,), PAD_NODE, jnp.int32)]).reshape(CHUNKS_PAD, CH)
    cidx = jnp.concatenate(
        [col, jnp.zeros((EP - E,), jnp.int32)]).reshape(CHUNKS_PAD, CH)
    ridx_g = jnp.where(ridx >= N, 0, ridx)

    zr = jnp.broadcast_to(z[:, None, :], (B, A, L)).reshape(N, L)
    atp = jnp.pad(atom_types.astype(f32), ((0, 0), (0, 6)))
    wapp = jnp.pad(Wap.astype(f32), ((0, 6), (0, 0)))
    c16 = jnp.pad(
        jnp.broadcast_to(init_coords[None], (B, A, 3)).reshape(N, 3),
        ((0, 0), (0, 13)))

    blp2 = blp.reshape(1, H)
    bap2 = bap.reshape(1, H)
    sel = jnp.zeros((4, H), f32).at[0, 0].set(1.0).at[1, 1].set(1.0).at[2, 2].set(1.0)
    lane3 = jnp.zeros((1, H), f32).at[0, 3].set(1.0)
    zeros_big = jnp.zeros((NPAD, H), f32)
    hw2p = jnp.pad(hW2, ((0, 0), (0, 13)))
    hb2p = jnp.pad(hb2.reshape(1, 3), ((0, 0), (0, 13)))

    nf, P, Q = _precompute(zr, atp, Wlp, blp2, wapp, bap2,
                           eW1[0, :H], eW1[0, H:2 * H], eb1[0].reshape(1, H))

    for i in range(NL):
        c16p = jnp.pad(c16, ((0, NPAD - N), (0, 0)))
        cx, cy, cz = c16p[:, 0], c16p[:, 1], c16p[:, 2]
        gp, gq, relt = _sc_gather()(P, Q, cx, cy, cz, ridx_g, cidx)
        rel4 = relt.T
        m, s128 = _edge_mlp(gp, gq, rel4,
                            eW1[i, 2 * H].reshape(1, H), eW2[i],
                            eb2[i].reshape(1, H), cW1[i],
                            cb1[i].reshape(1, H), cW2[i].reshape(1, H),
                            sel, lane3)
        agg, cu128 = _sc_scatter()(m, s128, ridx, zeros_big)
        aggN = agg[:N]
        cu16 = cu128[:N, :16]
        if i < NL - 1:
            nf, c16, P, Q = _node_mid(
                nf, aggN, cu16, c16,
                nW1[i, :H], nW1[i, H:], nb1[i].reshape(1, H),
                nW2[i], nb2[i].reshape(1, H),
                gamma[i].reshape(1, H), beta[i].reshape(1, H),
                eW1[i + 1, :H], eW1[i + 1, H:2 * H], eb1[i + 1].reshape(1, H))
        else:
            out16 = _node_final(
                nf, aggN, cu16, c16,
                nW1[i, :H], nW1[i, H:], nb1[i].reshape(1, H),
                nW2[i], nb2[i].reshape(1, H),
                gamma[i].reshape(1, H), beta[i].reshape(1, H),
                hW1, hb1.reshape(1, H // 2), hw2p, hb2p)
    return out16[:, :3]


# speedup vs baseline: 3.2496x; 1.1324x over previous
import functools

import jax
import jax.numpy as jnp
from jax import lax
from jax.experimental import pallas as pl
from jax.experimental.pallas import tpu as pltpu
from jax.experimental.pallas import tpu_sc as plsc

B, A, H, L, F, E = 200, 50, 128, 128, 10, 320000
N = B * A
NL = 2

NC, NS = 2, 16
NW = NC * NS
CH = 128
CHUNKS = -(-E // CH)
KPW = 2 * (-(-(-(-CHUNKS // NW)) // 2))
CHUNKS_PAD = NW * KPW
KPT = CHUNKS_PAD // NS
EP = CHUNKS_PAD * CH
NPAD = 10240
ROWS_PER_TILE = NPAD // NS
PAD_NODE = N + 16

BE = 2560
GRID_E = EP // BE


def _precompute_body(zr_ref, at_ref, wlp_ref, blp_ref, wap_ref, bap_ref,
                     w1a_ref, w1b_ref, eb1_ref,
                     nf_ref, p_ref, q_ref):
    nf = (jnp.dot(zr_ref[...], wlp_ref[...], preferred_element_type=jnp.float32)
          + blp_ref[...]
          + jnp.dot(at_ref[...], wap_ref[...], preferred_element_type=jnp.float32)
          + bap_ref[...])
    nf_ref[...] = nf
    p_ref[...] = jnp.dot(nf, w1a_ref[...], preferred_element_type=jnp.float32) + eb1_ref[...]
    q_ref[...] = jnp.dot(nf, w1b_ref[...], preferred_element_type=jnp.float32)


def _precompute(zr, atp, wlp, blp, wapp, bap, w1a, w1b, eb1):
    return pl.pallas_call(
        _precompute_body,
        out_shape=[jax.ShapeDtypeStruct((N, H), jnp.float32)] * 3,
    )(zr, atp, wlp, blp, wapp, bap, w1a, w1b, eb1)


def _edge_body(gp_ref, gq_ref, rel_ref, wd_ref, ew2_ref, eb2_ref,
               cw1_ref, cb1_ref, cw2_ref, sel_ref, lane3_ref,
               m_ref, s_ref):
    rel = rel_ref[...]
    dist = rel[:, 3:4]
    g = gp_ref[...] + gq_ref[...] + dist * wd_ref[...]
    m1 = jax.nn.silu(g)
    m = jax.nn.silu(
        jnp.dot(m1, ew2_ref[...], preferred_element_type=jnp.float32)
        + eb2_ref[...])
    cwp = jax.nn.silu(
        jnp.dot(m, cw1_ref[...], preferred_element_type=jnp.float32)
        + cb1_ref[...])
    cw = jnp.clip(jnp.sum(cwp * cw2_ref[...], axis=-1, keepdims=True), -1.0, 1.0)
    m_ref[...] = m
    s_ref[...] = cw * jnp.dot(rel, sel_ref[...],
                              preferred_element_type=jnp.float32) + lane3_ref[...]


def _edge_mlp(gp, gq, rel4, wd, ew2, eb2, cw1, cb1, cw2r, sel, lane3):
    full128 = pl.BlockSpec((H, H), lambda i: (0, 0))
    row128 = pl.BlockSpec((1, H), lambda i: (0, 0))
    return pl.pallas_call(
        _edge_body,
        grid=(GRID_E,),
        in_specs=[
            pl.BlockSpec((BE, H), lambda i: (i, 0)),
            pl.BlockSpec((BE, H), lambda i: (i, 0)),
            pl.BlockSpec((BE, 4), lambda i: (i, 0)),
            row128, full128, row128, full128, row128, row128,
            pl.BlockSpec((4, H), lambda i: (0, 0)), row128,
        ],
        out_specs=[
            pl.BlockSpec((BE, H), lambda i: (i, 0)),
            pl.BlockSpec((BE, H), lambda i: (i, 0)),
        ],
        out_shape=[
            jax.ShapeDtypeStruct((EP, H), jnp.float32),
            jax.ShapeDtypeStruct((EP, H), jnp.float32),
        ],
    )(gp, gq, rel4, wd, ew2, eb2, cw1, cb1, cw2r, sel, lane3)


def _layernorm(xn, gamma, beta):
    mu = jnp.mean(xn, axis=-1, keepdims=True)
    var = jnp.mean((xn - mu) ** 2, axis=-1, keepdims=True)
    return (xn - mu) * jax.lax.rsqrt(var + 1e-5) * gamma + beta


def _node_mid_body(nf_ref, agg_ref, cu_ref, c_ref,
                   nw1a_ref, nw1b_ref, nb1_ref, nw2_ref, nb2_ref,
                   gam_ref, bet_ref, w1a_ref, w1b_ref, eb1_ref,
                   nf2_ref, c2_ref, p_ref, q_ref):
    nf = nf_ref[...]
    agg = agg_ref[...]
    xn = jax.nn.silu(
        jnp.dot(nf, nw1a_ref[...], preferred_element_type=jnp.float32)
        + jnp.dot(agg, nw1b_ref[...], preferred_element_type=jnp.float32)
        + nb1_ref[...])
    xn = jnp.dot(xn, nw2_ref[...], preferred_element_type=jnp.float32) + nb2_ref[...]
    xn = _layernorm(xn, gam_ref[...], bet_ref[...])
    nf2_ref[...] = xn
    cu = cu_ref[...]
    c2_ref[...] = c_ref[...] + cu / (cu[:, 3:4] + 1e-6)
    p_ref[...] = jnp.dot(xn, w1a_ref[...], preferred_element_type=jnp.float32) + eb1_ref[...]
    q_ref[...] = jnp.dot(xn, w1b_ref[...], preferred_element_type=jnp.float32)


def _node_mid(nf, agg, cu16, c16, nw1a, nw1b, nb1, nw2, nb2,
              gam, bet, w1a, w1b, eb1):
    return pl.pallas_call(
        _node_mid_body,
        out_shape=[
            jax.ShapeDtypeStruct((N, H), jnp.float32),
            jax.ShapeDtypeStruct((N, 16), jnp.float32),
            jax.ShapeDtypeStruct((N, H), jnp.float32),
            jax.ShapeDtypeStruct((N, H), jnp.float32),
        ],
    )(nf, agg, cu16, c16, nw1a, nw1b, nb1, nw2, nb2, gam, bet, w1a, w1b, eb1)


def _node_final_body(nf_ref, agg_ref, cu_ref, c_ref,
                     nw1a_ref, nw1b_ref, nb1_ref, nw2_ref, nb2_ref,
                     gam_ref, bet_ref, hw1_ref, hb1_ref, hw2_ref, hb2_ref,
                     out_ref):
    nf = nf_ref[...]
    agg = agg_ref[...]
    xn = jax.nn.silu(
        jnp.dot(nf, nw1a_ref[...], preferred_element_type=jnp.float32)
        + jnp.dot(agg, nw1b_ref[...], preferred_element_type=jnp.float32)
        + nb1_ref[...])
    xn = jnp.dot(xn, nw2_ref[...], preferred_element_type=jnp.float32) + nb2_ref[...]
    xn = _layernorm(xn, gam_ref[...], bet_ref[...])
    cu = cu_ref[...]
    c2 = c_ref[...] + cu / (cu[:, 3:4] + 1e-6)
    hid = jax.nn.silu(
        jnp.dot(xn, hw1_ref[...], preferred_element_type=jnp.float32)
        + hb1_ref[...])
    cd = jnp.dot(hid, hw2_ref[...], preferred_element_type=jnp.float32) + hb2_ref[...]
    out_ref[...] = c2 + cd


def _node_final(nf, agg, cu16, c16, nw1a, nw1b, nb1, nw2, nb2,
                gam, bet, hw1, hb1, hw2p, hb2p):
    return pl.pallas_call(
        _node_final_body,
        out_shape=jax.ShapeDtypeStruct((N, 16), jnp.float32),
    )(nf, agg, cu16, c16, nw1a, nw1b, nb1, nw2, nb2, gam, bet,
      hw1, hb1, hw2p, hb2p)


@functools.cache
def _mesh():
    return plsc.VectorSubcoreMesh(core_axis_name="c", subcore_axis_name="s",
                                  num_cores=NC, num_subcores=NS)


def _sc_gather_body(p_hbm, q_hbm, cx_hbm, cy_hbm, cz_hbm, ridx_hbm, cidx_hbm,
                    gp_hbm, gq_hbm, relt_hbm,
                    rv0, cv0, rv1, cv1, gpb0, gqb0, gpb1, gqb1,
                    rxb, ryb, rzb, db, cxv, cyv, czv,
                    sa0, sb0, sa1, sb1):
    wid = lax.axis_index("s") * NC + lax.axis_index("c")
    base = wid * KPW
    pltpu.sync_copy(cx_hbm, cxv)
    pltpu.sync_copy(cy_hbm, cyv)
    pltpu.sync_copy(cz_hbm, czv)

    def fire(j, rv, cv, gpb, gqb, sa, sb):
        pltpu.sync_copy(ridx_hbm.at[j], rv)
        pltpu.sync_copy(cidx_hbm.at[j], cv)
        pltpu.async_copy(p_hbm.at[rv], gpb, sa)
        pltpu.async_copy(q_hbm.at[cv], gqb, sb)

    def process(j, rv, cv, gpb, gqb, sa, sb):
        pltpu.make_async_copy(p_hbm.at[rv], gpb, sa).wait()
        pltpu.make_async_copy(q_hbm.at[cv], gqb, sb).wait()
        for v in range(CH // 16):
            sl = pl.ds(v * 16, 16)
            ir = rv[sl]
            ic = cv[sl]
            rx = plsc.load_gather(cxv, [ir]) - plsc.load_gather(cxv, [ic])
            ry = plsc.load_gather(cyv, [ir]) - plsc.load_gather(cyv, [ic])
            rz = plsc.load_gather(czv, [ir]) - plsc.load_gather(czv, [ic])
            d = jnp.clip(rx * rx + ry * ry + rz * rz, 1e-6, 1e6)
            rxb[sl] = rx
            ryb[sl] = ry
            rzb[sl] = rz
            db[sl] = d
        eb = j * CH
        pltpu.sync_copy(gpb, gp_hbm.at[pl.ds(eb, CH)])
        pltpu.sync_copy(gqb, gq_hbm.at[pl.ds(eb, CH)])
        pltpu.sync_copy(rxb, relt_hbm.at[0, pl.ds(eb, CH)])
        pltpu.sync_copy(ryb, relt_hbm.at[1, pl.ds(eb, CH)])
        pltpu.sync_copy(rzb, relt_hbm.at[2, pl.ds(eb, CH)])
        pltpu.sync_copy(db, relt_hbm.at[3, pl.ds(eb, CH)])

    fire(base, rv0, cv0, gpb0, gqb0, sa0, sb0)
    KH = KPW // 2

    def body(k, _):
        j0 = base + 2 * k
        fire(j0 + 1, rv1, cv1, gpb1, gqb1, sa1, sb1)
        process(j0, rv0, cv0, gpb0, gqb0, sa0, sb0)

        @pl.when(k < KH - 1)
        def _f():
            fire(j0 + 2, rv0, cv0, gpb0, gqb0, sa0, sb0)

        process(j0 + 1, rv1, cv1, gpb1, gqb1, sa1, sb1)
        return _

    lax.fori_loop(0, KH, body, None)


@functools.cache
def _sc_gather():
    return pl.kernel(
        _sc_gather_body,
        out_type=[
            jax.ShapeDtypeStruct((EP, H), jnp.float32),
            jax.ShapeDtypeStruct((EP, H), jnp.float32),
            jax.ShapeDtypeStruct((4, EP), jnp.float32),
        ],
        mesh=_mesh(),
        scratch_types=[
            pltpu.VMEM((CH,), jnp.int32),
            pltpu.VMEM((CH,), jnp.int32),
            pltpu.VMEM((CH,), jnp.int32),
            pltpu.VMEM((CH,), jnp.int32),
            pltpu.VMEM((CH, H), jnp.float32),
            pltpu.VMEM((CH, H), jnp.float32),
            pltpu.VMEM((CH, H), jnp.float32),
            pltpu.VMEM((CH, H), jnp.float32),
            pltpu.VMEM((CH,), jnp.float32),
            pltpu.VMEM((CH,), jnp.float32),
            pltpu.VMEM((CH,), jnp.float32),
            pltpu.VMEM((CH,), jnp.float32),
            pltpu.VMEM((NPAD,), jnp.float32),
            pltpu.VMEM((NPAD,), jnp.float32),
            pltpu.VMEM((NPAD,), jnp.float32),
            pltpu.SemaphoreType.DMA,
            pltpu.SemaphoreType.DMA,
            pltpu.SemaphoreType.DMA,
            pltpu.SemaphoreType.DMA,
        ],
        compiler_params=pltpu.CompilerParams(needs_layout_passes=False),
    )


def _sc_scatter_body(m_hbm, s_hbm, ridx_hbm, zero_hbm,
                     agg_hbm, cu_hbm,
                     rv0, rv1, mb0, mb1, acc_sh, sm0, sm1):
    cid = lax.axis_index("c")
    sid = lax.axis_index("s")
    rows = pl.ds(sid * ROWS_PER_TILE, ROWS_PER_TILE)
    pltpu.sync_copy(zero_hbm.at[rows], acc_sh.at[rows])
    plsc.subcore_barrier()
    base = sid * KPT
    KH = KPT // 2

    def run(src_hbm):
        def fire(j, rv, mb, sm):
            pltpu.sync_copy(ridx_hbm.at[j], rv)
            pltpu.async_copy(src_hbm.at[pl.ds(j * CH, CH)], mb, sm)

        def proc(rv, mb, sm):
            pltpu.make_async_copy(src_hbm.at[pl.ds(0, CH)], mb, sm).wait()
            pltpu.sync_copy(mb, acc_sh.at[rv], add=True)

        fire(base, rv0, mb0, sm0)

        def body(k, carry):
            j0 = base + 2 * k
            fire(j0 + 1, rv1, mb1, sm1)
            proc(rv0, mb0, sm0)

            @pl.when(k < KH - 1)
            def _f():
                fire(j0 + 2, rv0, mb0, sm0)

            proc(rv1, mb1, sm1)
            return carry

        lax.fori_loop(0, KH, body, None)

    @pl.when(cid == 0)
    def _():
        run(m_hbm)

    @pl.when(cid == 1)
    def _():
        run(s_hbm)

    plsc.subcore_barrier()

    @pl.when(cid == 0)
    def _():
        pltpu.sync_copy(acc_sh.at[rows], agg_hbm.at[rows])

    @pl.when(cid == 1)
    def _():
        pltpu.sync_copy(acc_sh.at[rows], cu_hbm.at[rows])


@functools.cache
def _sc_scatter():
    return pl.kernel(
        _sc_scatter_body,
        out_type=[
            jax.ShapeDtypeStruct((NPAD, H), jnp.float32),
            jax.ShapeDtypeStruct((NPAD, H), jnp.float32),
        ],
        mesh=_mesh(),
        scratch_types=[
            pltpu.VMEM((CH,), jnp.int32),
            pltpu.VMEM((CH,), jnp.int32),
            pltpu.VMEM((CH, H), jnp.float32),
            pltpu.VMEM((CH, H), jnp.float32),
            pltpu.VMEM_SHARED((NPAD, H), jnp.float32),
            pltpu.SemaphoreType.DMA,
            pltpu.SemaphoreType.DMA,
        ],
    )


def kernel(z, atom_types, edge_index, Wlp, blp, Wap, bap, init_coords,
           eW1, eb1, eW2, eb2, nW1, nb1, nW2, nb2, cW1, cb1, cW2,
           gamma, beta, hW1, hb1, hW2, hb2):
    f32 = jnp.float32
    row = edge_index[0].astype(jnp.int32)
    col = edge_index[1].astype(jnp.int32)
    ridx = jnp.concatenate(
        [row, jnp.full((EP - E,), PAD_NODE, jnp.int32)]).reshape(CHUNKS_PAD, CH)
    cidx = jnp.concatenate(
        [col, jnp.zeros((EP - E,), jnp.int32)]).reshape(CHUNKS_PAD, CH)
    ridx_g = jnp.where(ridx >= N, 0, ridx)

    zr = jnp.broadcast_to(z[:, None, :], (B, A, L)).reshape(N, L)
    atp = jnp.pad(atom_types.astype(f32), ((0, 0), (0, 6)))
    wapp = jnp.pad(Wap.astype(f32), ((0, 6), (0, 0)))
    c16 = jnp.pad(
        jnp.broadcast_to(init_coords[None], (B, A, 3)).reshape(N, 3),
        ((0, 0), (0, 13)))

    blp2 = blp.reshape(1, H)
    bap2 = bap.reshape(1, H)
    sel = jnp.zeros((4, H), f32).at[0, 0].set(1.0).at[1, 1].set(1.0).at[2, 2].set(1.0)
    lane3 = jnp.zeros((1, H), f32).at[0, 3].set(1.0)
    zeros_big = jnp.zeros((NPAD, H), f32)
    hw2p = jnp.pad(hW2, ((0, 0), (0, 13)))
    hb2p = jnp.pad(hb2.reshape(1, 3), ((0, 0), (0, 13)))

    nf, P, Q = _precompute(zr, atp, Wlp, blp2, wapp, bap2,
                           eW1[0, :H], eW1[0, H:2 * H], eb1[0].reshape(1, H))

    for i in range(NL):
        c16p = jnp.pad(c16, ((0, NPAD - N), (0, 0)))
        cx, cy, cz = c16p[:, 0], c16p[:, 1], c16p[:, 2]
        gp, gq, relt = _sc_gather()(P, Q, cx, cy, cz, ridx_g, cidx)
        rel4 = relt.T
        m, s128 = _edge_mlp(gp, gq, rel4,
                            eW1[i, 2 * H].reshape(1, H), eW2[i],
                            eb2[i].reshape(1, H), cW1[i],
                            cb1[i].reshape(1, H), cW2[i].reshape(1, H),
                            sel, lane3)
        agg, cu128 = _sc_scatter()(m, s128, ridx, zeros_big)
        aggN = agg[:N]
        cu16 = cu128[:N, :16]
        if i < NL - 1:
            nf, c16, P, Q = _node_mid(
                nf, aggN, cu16, c16,
                nW1[i, :H], nW1[i, H:], nb1[i].reshape(1, H),
                nW2[i], nb2[i].reshape(1, H),
                gamma[i].reshape(1, H), beta[i].reshape(1, H),
                eW1[i + 1, :H], eW1[i + 1, H:2 * H], eb1[i + 1].reshape(1, H))
        else:
            out16 = _node_final(
                nf, aggN, cu16, c16,
                nW1[i, :H], nW1[i, H:], nb1[i].reshape(1, H),
                nW2[i], nb2[i].reshape(1, H),
                gamma[i].reshape(1, H), beta[i].reshape(1, H),
                hW1, hb1.reshape(1, H // 2), hw2p, hb2p)
    return out16[:, :3]


# speedup vs baseline: 3.6584x; 1.1258x over previous
import functools

import jax
import jax.numpy as jnp
from jax import lax
from jax.experimental import pallas as pl
from jax.experimental.pallas import tpu as pltpu
from jax.experimental.pallas import tpu_sc as plsc

B, A, H, L, F, E = 200, 50, 128, 128, 10, 320000
N = B * A
NL = 2

NC, NS = 2, 16
NW = NC * NS
CH = 128
CHUNKS = -(-E // CH)
KPW = 2 * (-(-(-(-CHUNKS // NW)) // 2))
CHUNKS_PAD = NW * KPW
KPT = CHUNKS_PAD // NS
EP = CHUNKS_PAD * CH
NPAD = 10240
ROWS_PER_TILE = NPAD // NS
PAD_NODE = N + 16

BE = 2560
GRID_E = EP // BE


def _precompute_body(zr_ref, at_ref, wlp_ref, blp_ref, wap_ref, bap_ref,
                     w1a_ref, w1b_ref, eb1_ref,
                     nf_ref, p_ref, q_ref):
    nf = (jnp.dot(zr_ref[...], wlp_ref[...], preferred_element_type=jnp.float32)
          + blp_ref[...]
          + jnp.dot(at_ref[...], wap_ref[...], preferred_element_type=jnp.float32)
          + bap_ref[...])
    nf_ref[...] = nf
    p_ref[...] = jnp.dot(nf, w1a_ref[...], preferred_element_type=jnp.float32) + eb1_ref[...]
    q_ref[...] = jnp.dot(nf, w1b_ref[...], preferred_element_type=jnp.float32)


def _precompute(zr, atp, wlp, blp, wapp, bap, w1a, w1b, eb1):
    return pl.pallas_call(
        _precompute_body,
        out_shape=[jax.ShapeDtypeStruct((N, H), jnp.float32)] * 3,
    )(zr, atp, wlp, blp, wapp, bap, w1a, w1b, eb1)


def _edge_body(gp_ref, rel_ref, wd_ref, ew2_ref, eb2_ref,
               cw1_ref, cb1_ref, cw2_ref, sel_ref, lane3_ref,
               m_ref, s_ref):
    rel = jnp.transpose(rel_ref[...], (1, 0))
    dist = rel[:, 3:4]
    g = gp_ref[...] + dist * wd_ref[...]
    m1 = jax.nn.silu(g)
    m = jax.nn.silu(
        jnp.dot(m1, ew2_ref[...], preferred_element_type=jnp.float32)
        + eb2_ref[...])
    cwp = jax.nn.silu(
        jnp.dot(m, cw1_ref[...], preferred_element_type=jnp.float32)
        + cb1_ref[...])
    cw = jnp.clip(jnp.sum(cwp * cw2_ref[...], axis=-1, keepdims=True), -1.0, 1.0)
    m_ref[...] = m
    s_ref[...] = cw * jnp.dot(rel, sel_ref[...],
                              preferred_element_type=jnp.float32) + lane3_ref[...]


def _edge_mlp(gp, relt, wd, ew2, eb2, cw1, cb1, cw2r, sel, lane3):
    full128 = pl.BlockSpec((H, H), lambda i: (0, 0))
    row128 = pl.BlockSpec((1, H), lambda i: (0, 0))
    return pl.pallas_call(
        _edge_body,
        grid=(GRID_E,),
        in_specs=[
            pl.BlockSpec((BE, H), lambda i: (i, 0)),
            pl.BlockSpec((4, BE), lambda i: (0, i)),
            row128, full128, row128, full128, row128, row128,
            pl.BlockSpec((4, H), lambda i: (0, 0)), row128,
        ],
        out_specs=[
            pl.BlockSpec((BE, H), lambda i: (i, 0)),
            pl.BlockSpec((BE, H), lambda i: (i, 0)),
        ],
        out_shape=[
            jax.ShapeDtypeStruct((EP, H), jnp.float32),
            jax.ShapeDtypeStruct((EP, H), jnp.float32),
        ],
    )(gp, relt, wd, ew2, eb2, cw1, cb1, cw2r, sel, lane3)


def _layernorm(xn, gamma, beta):
    mu = jnp.mean(xn, axis=-1, keepdims=True)
    var = jnp.mean((xn - mu) ** 2, axis=-1, keepdims=True)
    return (xn - mu) * jax.lax.rsqrt(var + 1e-5) * gamma + beta


def _node_mid_body(nf_ref, agg_ref, cu_ref, c_ref,
                   nw1a_ref, nw1b_ref, nb1_ref, nw2_ref, nb2_ref,
                   gam_ref, bet_ref, w1a_ref, w1b_ref, eb1_ref,
                   nf2_ref, c2_ref, p_ref, q_ref):
    nf = nf_ref[...]
    agg = agg_ref[...]
    xn = jax.nn.silu(
        jnp.dot(nf, nw1a_ref[...], preferred_element_type=jnp.float32)
        + jnp.dot(agg, nw1b_ref[...], preferred_element_type=jnp.float32)
        + nb1_ref[...])
    xn = jnp.dot(xn, nw2_ref[...], preferred_element_type=jnp.float32) + nb2_ref[...]
    xn = _layernorm(xn, gam_ref[...], bet_ref[...])
    nf2_ref[...] = xn
    cu = cu_ref[...]
    c2_ref[...] = c_ref[...] + cu / (cu[:, 3:4] + 1e-6)
    p_ref[...] = jnp.dot(xn, w1a_ref[...], preferred_element_type=jnp.float32) + eb1_ref[...]
    q_ref[...] = jnp.dot(xn, w1b_ref[...], preferred_element_type=jnp.float32)


def _node_mid(nf, agg, cu16, c16, nw1a, nw1b, nb1, nw2, nb2,
              gam, bet, w1a, w1b, eb1):
    return pl.pallas_call(
        _node_mid_body,
        out_shape=[
            jax.ShapeDtypeStruct((N, H), jnp.float32),
            jax.ShapeDtypeStruct((N, 16), jnp.float32),
            jax.ShapeDtypeStruct((N, H), jnp.float32),
            jax.ShapeDtypeStruct((N, H), jnp.float32),
        ],
    )(nf, agg, cu16, c16, nw1a, nw1b, nb1, nw2, nb2, gam, bet, w1a, w1b, eb1)


def _node_final_body(nf_ref, agg_ref, cu_ref, c_ref,
                     nw1a_ref, nw1b_ref, nb1_ref, nw2_ref, nb2_ref,
                     gam_ref, bet_ref, hw1_ref, hb1_ref, hw2_ref, hb2_ref,
                     out_ref):
    nf = nf_ref[...]
    agg = agg_ref[...]
    xn = jax.nn.silu(
        jnp.dot(nf, nw1a_ref[...], preferred_element_type=jnp.float32)
        + jnp.dot(agg, nw1b_ref[...], preferred_element_type=jnp.float32)
        + nb1_ref[...])
    xn = jnp.dot(xn, nw2_ref[...], preferred_element_type=jnp.float32) + nb2_ref[...]
    xn = _layernorm(xn, gam_ref[...], bet_ref[...])
    cu = cu_ref[...]
    c2 = c_ref[...] + cu / (cu[:, 3:4] + 1e-6)
    hid = jax.nn.silu(
        jnp.dot(xn, hw1_ref[...], preferred_element_type=jnp.float32)
        + hb1_ref[...])
    cd = jnp.dot(hid, hw2_ref[...], preferred_element_type=jnp.float32) + hb2_ref[...]
    out_ref[...] = c2 + cd


def _node_final(nf, agg, cu16, c16, nw1a, nw1b, nb1, nw2, nb2,
                gam, bet, hw1, hb1, hw2p, hb2p):
    return pl.pallas_call(
        _node_final_body,
        out_shape=jax.ShapeDtypeStruct((N, 16), jnp.float32),
    )(nf, agg, cu16, c16, nw1a, nw1b, nb1, nw2, nb2, gam, bet,
      hw1, hb1, hw2p, hb2p)


@functools.cache
def _mesh():
    return plsc.VectorSubcoreMesh(core_axis_name="c", subcore_axis_name="s",
                                  num_cores=NC, num_subcores=NS)


def _sc_gather_body(p_hbm, q_hbm, cx_hbm, cy_hbm, cz_hbm, ridx_hbm, cidx_hbm,
                    gp_hbm, relt_hbm,
                    rv0, cv0, rv1, cv1, gpb0, gqb0, gpb1, gqb1,
                    rxb, ryb, rzb, db, cxv, cyv, czv,
                    sa0, sb0, sa1, sb1):
    wid = lax.axis_index("s") * NC + lax.axis_index("c")
    base = wid * KPW
    pltpu.sync_copy(cx_hbm, cxv)
    pltpu.sync_copy(cy_hbm, cyv)
    pltpu.sync_copy(cz_hbm, czv)

    def fire(j, rv, cv, gpb, gqb, sa, sb):
        pltpu.sync_copy(ridx_hbm.at[j], rv)
        pltpu.sync_copy(cidx_hbm.at[j], cv)
        pltpu.async_copy(p_hbm.at[rv], gpb, sa)
        pltpu.async_copy(q_hbm.at[cv], gqb, sb)

    def process(j, rv, cv, gpb, gqb, sa, sb):
        pltpu.make_async_copy(p_hbm.at[rv], gpb, sa).wait()
        pltpu.make_async_copy(q_hbm.at[cv], gqb, sb).wait()
        for v in range(CH // 16):
            sl = pl.ds(v * 16, 16)
            ir = rv[sl]
            ic = cv[sl]
            rx = plsc.load_gather(cxv, [ir]) - plsc.load_gather(cxv, [ic])
            ry = plsc.load_gather(cyv, [ir]) - plsc.load_gather(cyv, [ic])
            rz = plsc.load_gather(czv, [ir]) - plsc.load_gather(czv, [ic])
            d = jnp.clip(rx * rx + ry * ry + rz * rz, 1e-6, 1e6)
            rxb[sl] = rx
            ryb[sl] = ry
            rzb[sl] = rz
            db[sl] = d

        def addrow(r, _):
            for u in range(H // 16):
                su = pl.ds(u * 16, 16)
                gpb[r, su] = gpb[r, su] + gqb[r, su]
            return _
        lax.fori_loop(0, CH, addrow, None)
        eb = j * CH
        pltpu.sync_copy(gpb, gp_hbm.at[pl.ds(eb, CH)])
        pltpu.sync_copy(rxb, relt_hbm.at[0, pl.ds(eb, CH)])
        pltpu.sync_copy(ryb, relt_hbm.at[1, pl.ds(eb, CH)])
        pltpu.sync_copy(rzb, relt_hbm.at[2, pl.ds(eb, CH)])
        pltpu.sync_copy(db, relt_hbm.at[3, pl.ds(eb, CH)])

    fire(base, rv0, cv0, gpb0, gqb0, sa0, sb0)
    KH = KPW // 2

    def body(k, _):
        j0 = base + 2 * k
        fire(j0 + 1, rv1, cv1, gpb1, gqb1, sa1, sb1)
        process(j0, rv0, cv0, gpb0, gqb0, sa0, sb0)

        @pl.when(k < KH - 1)
        def _f():
            fire(j0 + 2, rv0, cv0, gpb0, gqb0, sa0, sb0)

        process(j0 + 1, rv1, cv1, gpb1, gqb1, sa1, sb1)
        return _

    lax.fori_loop(0, KH, body, None)


@functools.cache
def _sc_gather():
    return pl.kernel(
        _sc_gather_body,
        out_type=[
            jax.ShapeDtypeStruct((EP, H), jnp.float32),
            jax.ShapeDtypeStruct((4, EP), jnp.float32),
        ],
        mesh=_mesh(),
        scratch_types=[
            pltpu.VMEM((CH,), jnp.int32),
            pltpu.VMEM((CH,), jnp.int32),
            pltpu.VMEM((CH,), jnp.int32),
            pltpu.VMEM((CH,), jnp.int32),
            pltpu.VMEM((CH, H), jnp.float32),
            pltpu.VMEM((CH, H), jnp.float32),
            pltpu.VMEM((CH, H), jnp.float32),
            pltpu.VMEM((CH, H), jnp.float32),
            pltpu.VMEM((CH,), jnp.float32),
            pltpu.VMEM((CH,), jnp.float32),
            pltpu.VMEM((CH,), jnp.float32),
            pltpu.VMEM((CH,), jnp.float32),
            pltpu.VMEM((NPAD,), jnp.float32),
            pltpu.VMEM((NPAD,), jnp.float32),
            pltpu.VMEM((NPAD,), jnp.float32),
            pltpu.SemaphoreType.DMA,
            pltpu.SemaphoreType.DMA,
            pltpu.SemaphoreType.DMA,
            pltpu.SemaphoreType.DMA,
        ],
        compiler_params=pltpu.CompilerParams(needs_layout_passes=False),
    )


def _sc_scatter_body(m_hbm, s_hbm, ridx_hbm, zero_hbm,
                     agg_hbm, cu_hbm,
                     rv0, rv1, mb0, mb1, acc_sh, sm0, sm1):
    cid = lax.axis_index("c")
    sid = lax.axis_index("s")
    rows = pl.ds(sid * ROWS_PER_TILE, ROWS_PER_TILE)
    pltpu.sync_copy(zero_hbm.at[rows], acc_sh.at[rows])
    plsc.subcore_barrier()
    base = sid * KPT
    KH = KPT // 2

    def run(src_hbm):
        def fire(j, rv, mb, sm):
            pltpu.sync_copy(ridx_hbm.at[j], rv)
            pltpu.async_copy(src_hbm.at[pl.ds(j * CH, CH)], mb, sm)

        def proc(rv, mb, sm):
            pltpu.make_async_copy(src_hbm.at[pl.ds(0, CH)], mb, sm).wait()
            pltpu.sync_copy(mb, acc_sh.at[rv], add=True)

        fire(base, rv0, mb0, sm0)

        def body(k, carry):
            j0 = base + 2 * k
            fire(j0 + 1, rv1, mb1, sm1)
            proc(rv0, mb0, sm0)

            @pl.when(k < KH - 1)
            def _f():
                fire(j0 + 2, rv0, mb0, sm0)

            proc(rv1, mb1, sm1)
            return carry

        lax.fori_loop(0, KH, body, None)

    @pl.when(cid == 0)
    def _():
        run(m_hbm)

    @pl.when(cid == 1)
    def _():
        run(s_hbm)

    plsc.subcore_barrier()

    @pl.when(cid == 0)
    def _():
        pltpu.sync_copy(acc_sh.at[rows], agg_hbm.at[rows])

    @pl.when(cid == 1)
    def _():
        pltpu.sync_copy(acc_sh.at[rows], cu_hbm.at[rows])


@functools.cache
def _sc_scatter():
    return pl.kernel(
        _sc_scatter_body,
        out_type=[
            jax.ShapeDtypeStruct((NPAD, H), jnp.float32),
            jax.ShapeDtypeStruct((NPAD, H), jnp.float32),
        ],
        mesh=_mesh(),
        scratch_types=[
            pltpu.VMEM((CH,), jnp.int32),
            pltpu.VMEM((CH,), jnp.int32),
            pltpu.VMEM((CH, H), jnp.float32),
            pltpu.VMEM((CH, H), jnp.float32),
            pltpu.VMEM_SHARED((NPAD, H), jnp.float32),
            pltpu.SemaphoreType.DMA,
            pltpu.SemaphoreType.DMA,
        ],
    )


def kernel(z, atom_types, edge_index, Wlp, blp, Wap, bap, init_coords,
           eW1, eb1, eW2, eb2, nW1, nb1, nW2, nb2, cW1, cb1, cW2,
           gamma, beta, hW1, hb1, hW2, hb2):
    f32 = jnp.float32
    row = edge_index[0].astype(jnp.int32)
    col = edge_index[1].astype(jnp.int32)
    ridx = jnp.concatenate(
        [row, jnp.full((EP - E,), PAD_NODE, jnp.int32)]).reshape(CHUNKS_PAD, CH)
    cidx = jnp.concatenate(
        [col, jnp.zeros((EP - E,), jnp.int32)]).reshape(CHUNKS_PAD, CH)
    ridx_g = jnp.where(ridx >= N, 0, ridx)

    zr = jnp.broadcast_to(z[:, None, :], (B, A, L)).reshape(N, L)
    atp = jnp.pad(atom_types.astype(f32), ((0, 0), (0, 6)))
    wapp = jnp.pad(Wap.astype(f32), ((0, 6), (0, 0)))
    c16 = jnp.pad(
        jnp.broadcast_to(init_coords[None], (B, A, 3)).reshape(N, 3),
        ((0, 0), (0, 13)))

    blp2 = blp.reshape(1, H)
    bap2 = bap.reshape(1, H)
    sel = jnp.zeros((4, H), f32).at[0, 0].set(1.0).at[1, 1].set(1.0).at[2, 2].set(1.0)
    lane3 = jnp.zeros((1, H), f32).at[0, 3].set(1.0)
    zeros_big = jnp.zeros((NPAD, H), f32)
    hw2p = jnp.pad(hW2, ((0, 0), (0, 13)))
    hb2p = jnp.pad(hb2.reshape(1, 3), ((0, 0), (0, 13)))

    nf, P, Q = _precompute(zr, atp, Wlp, blp2, wapp, bap2,
                           eW1[0, :H], eW1[0, H:2 * H], eb1[0].reshape(1, H))

    for i in range(NL):
        c16p = jnp.pad(c16, ((0, NPAD - N), (0, 0)))
        cx, cy, cz = c16p[:, 0], c16p[:, 1], c16p[:, 2]
        g, relt = _sc_gather()(P, Q, cx, cy, cz, ridx_g, cidx)
        m, s128 = _edge_mlp(g, relt,
                            eW1[i, 2 * H].reshape(1, H), eW2[i],
                            eb2[i].reshape(1, H), cW1[i],
                            cb1[i].reshape(1, H), cW2[i].reshape(1, H),
                            sel, lane3)
        agg, cu128 = _sc_scatter()(m, s128, ridx, zeros_big)
        aggN = agg[:N]
        cu16 = cu128[:N, :16]
        if i < NL - 1:
            nf, c16, P, Q = _node_mid(
                nf, aggN, cu16, c16,
                nW1[i, :H], nW1[i, H:], nb1[i].reshape(1, H),
                nW2[i], nb2[i].reshape(1, H),
                gamma[i].reshape(1, H), beta[i].reshape(1, H),
                eW1[i + 1, :H], eW1[i + 1, H:2 * H], eb1[i + 1].reshape(1, H))
        else:
            out16 = _node_final(
                nf, aggN, cu16, c16,
                nW1[i, :H], nW1[i, H:], nb1[i].reshape(1, H),
                nW2[i], nb2[i].reshape(1, H),
                gamma[i].reshape(1, H), beta[i].reshape(1, H),
                hW1, hb1.reshape(1, H // 2), hw2p, hb2p)
    return out16[:, :3]


# speedup vs baseline: 3.7608x; 1.0280x over previous
import functools

import jax
import jax.numpy as jnp
from jax import lax
from jax.experimental import pallas as pl
from jax.experimental.pallas import tpu as pltpu
from jax.experimental.pallas import tpu_sc as plsc

B, A, H, L, F, E = 200, 50, 128, 128, 10, 320000
N = B * A
NL = 2

NC, NS = 2, 16
NW = NC * NS
CH = 128
CHUNKS = -(-E // CH)
KPW = 2 * (-(-(-(-CHUNKS // NW)) // 2))
CHUNKS_PAD = NW * KPW
KPT = CHUNKS_PAD // NS
EP = CHUNKS_PAD * CH
NPAD = 10240
ROWS_PER_TILE = NPAD // NS
PAD_NODE = N + 16

BE = 5120
GRID_E = EP // BE


def _precompute_body(zr_ref, at_ref, wlp_ref, blp_ref, wap_ref, bap_ref,
                     w1a_ref, w1b_ref, eb1_ref,
                     nf_ref, p_ref, q_ref):
    nf = (jnp.dot(zr_ref[...], wlp_ref[...], preferred_element_type=jnp.float32)
          + blp_ref[...]
          + jnp.dot(at_ref[...], wap_ref[...], preferred_element_type=jnp.float32)
          + bap_ref[...])
    nf_ref[...] = nf
    p_ref[...] = jnp.dot(nf, w1a_ref[...], preferred_element_type=jnp.float32) + eb1_ref[...]
    q_ref[...] = jnp.dot(nf, w1b_ref[...], preferred_element_type=jnp.float32)


def _precompute(zr, atp, wlp, blp, wapp, bap, w1a, w1b, eb1):
    return pl.pallas_call(
        _precompute_body,
        out_shape=[jax.ShapeDtypeStruct((N, H), jnp.float32)] * 3,
    )(zr, atp, wlp, blp, wapp, bap, w1a, w1b, eb1)


def _edge_body(gp_ref, rel_ref, wd_ref, ew2_ref, eb2_ref,
               cw1_ref, cb1_ref, cw2_ref, sel_ref, lane3_ref,
               m_ref, s_ref):
    rel = jnp.transpose(rel_ref[...], (1, 0))
    dist = rel[:, 3:4]
    g = gp_ref[...] + dist * wd_ref[...]
    m1 = jax.nn.silu(g)
    m = jax.nn.silu(
        jnp.dot(m1, ew2_ref[...], preferred_element_type=jnp.float32)
        + eb2_ref[...])
    cwp = jax.nn.silu(
        jnp.dot(m, cw1_ref[...], preferred_element_type=jnp.float32)
        + cb1_ref[...])
    cw = jnp.clip(jnp.sum(cwp * cw2_ref[...], axis=-1, keepdims=True), -1.0, 1.0)
    m_ref[...] = m
    s_ref[...] = cw * jnp.dot(rel, sel_ref[...],
                              preferred_element_type=jnp.float32) + lane3_ref[...]


def _edge_mlp(gp, relt, wd, ew2, eb2, cw1, cb1, cw2r, sel, lane3):
    full128 = pl.BlockSpec((H, H), lambda i: (0, 0))
    row128 = pl.BlockSpec((1, H), lambda i: (0, 0))
    return pl.pallas_call(
        _edge_body,
        grid=(GRID_E,),
        in_specs=[
            pl.BlockSpec((BE, H), lambda i: (i, 0)),
            pl.BlockSpec((4, BE), lambda i: (0, i)),
            row128, full128, row128, full128, row128, row128,
            pl.BlockSpec((4, H), lambda i: (0, 0)), row128,
        ],
        out_specs=[
            pl.BlockSpec((BE, H), lambda i: (i, 0)),
            pl.BlockSpec((BE, H), lambda i: (i, 0)),
        ],
        out_shape=[
            jax.ShapeDtypeStruct((EP, H), jnp.float32),
            jax.ShapeDtypeStruct((EP, H), jnp.float32),
        ],
    )(gp, relt, wd, ew2, eb2, cw1, cb1, cw2r, sel, lane3)


def _layernorm(xn, gamma, beta):
    mu = jnp.mean(xn, axis=-1, keepdims=True)
    var = jnp.mean((xn - mu) ** 2, axis=-1, keepdims=True)
    return (xn - mu) * jax.lax.rsqrt(var + 1e-5) * gamma + beta


def _node_mid_body(nf_ref, agg_ref, cu_ref, c_ref,
                   nw1a_ref, nw1b_ref, nb1_ref, nw2_ref, nb2_ref,
                   gam_ref, bet_ref, w1a_ref, w1b_ref, eb1_ref,
                   nf2_ref, c2_ref, p_ref, q_ref):
    nf = nf_ref[...]
    agg = agg_ref[...]
    xn = jax.nn.silu(
        jnp.dot(nf, nw1a_ref[...], preferred_element_type=jnp.float32)
        + jnp.dot(agg, nw1b_ref[...], preferred_element_type=jnp.float32)
        + nb1_ref[...])
    xn = jnp.dot(xn, nw2_ref[...], preferred_element_type=jnp.float32) + nb2_ref[...]
    xn = _layernorm(xn, gam_ref[...], bet_ref[...])
    nf2_ref[...] = xn
    cu = cu_ref[...]
    c2_ref[...] = c_ref[...] + cu / (cu[:, 3:4] + 1e-6)
    p_ref[...] = jnp.dot(xn, w1a_ref[...], preferred_element_type=jnp.float32) + eb1_ref[...]
    q_ref[...] = jnp.dot(xn, w1b_ref[...], preferred_element_type=jnp.float32)


def _node_mid(nf, agg, cu16, c16, nw1a, nw1b, nb1, nw2, nb2,
              gam, bet, w1a, w1b, eb1):
    return pl.pallas_call(
        _node_mid_body,
        out_shape=[
            jax.ShapeDtypeStruct((N, H), jnp.float32),
            jax.ShapeDtypeStruct((N, 16), jnp.float32),
            jax.ShapeDtypeStruct((N, H), jnp.float32),
            jax.ShapeDtypeStruct((N, H), jnp.float32),
        ],
    )(nf, agg, cu16, c16, nw1a, nw1b, nb1, nw2, nb2, gam, bet, w1a, w1b, eb1)


def _node_final_body(nf_ref, agg_ref, cu_ref, c_ref,
                     nw1a_ref, nw1b_ref, nb1_ref, nw2_ref, nb2_ref,
                     gam_ref, bet_ref, hw1_ref, hb1_ref, hw2_ref, hb2_ref,
                     out_ref):
    nf = nf_ref[...]
    agg = agg_ref[...]
    xn = jax.nn.silu(
        jnp.dot(nf, nw1a_ref[...], preferred_element_type=jnp.float32)
        + jnp.dot(agg, nw1b_ref[...], preferred_element_type=jnp.float32)
        + nb1_ref[...])
    xn = jnp.dot(xn, nw2_ref[...], preferred_element_type=jnp.float32) + nb2_ref[...]
    xn = _layernorm(xn, gam_ref[...], bet_ref[...])
    cu = cu_ref[...]
    c2 = c_ref[...] + cu / (cu[:, 3:4] + 1e-6)
    hid = jax.nn.silu(
        jnp.dot(xn, hw1_ref[...], preferred_element_type=jnp.float32)
        + hb1_ref[...])
    cd = jnp.dot(hid, hw2_ref[...], preferred_element_type=jnp.float32) + hb2_ref[...]
    out_ref[...] = c2 + cd


def _node_final(nf, agg, cu16, c16, nw1a, nw1b, nb1, nw2, nb2,
                gam, bet, hw1, hb1, hw2p, hb2p):
    return pl.pallas_call(
        _node_final_body,
        out_shape=jax.ShapeDtypeStruct((N, 16), jnp.float32),
    )(nf, agg, cu16, c16, nw1a, nw1b, nb1, nw2, nb2, gam, bet,
      hw1, hb1, hw2p, hb2p)


@functools.cache
def _mesh():
    return plsc.VectorSubcoreMesh(core_axis_name="c", subcore_axis_name="s",
                                  num_cores=NC, num_subcores=NS)


def _sc_gather_body(p_hbm, q_hbm, cx_hbm, cy_hbm, cz_hbm, ridx_hbm, cidx_hbm,
                    gp_hbm, relt_hbm,
                    rv0, cv0, rv1, cv1, gpb0, gqb0, gpb1, gqb1,
                    rxb, ryb, rzb, db, cxv, cyv, czv,
                    sa0, sb0, sa1, sb1):
    wid = lax.axis_index("s") * NC + lax.axis_index("c")
    base = wid * KPW
    pltpu.sync_copy(cx_hbm, cxv)
    pltpu.sync_copy(cy_hbm, cyv)
    pltpu.sync_copy(cz_hbm, czv)

    def fire(j, rv, cv, gpb, gqb, sa, sb):
        pltpu.sync_copy(ridx_hbm.at[j], rv)
        pltpu.sync_copy(cidx_hbm.at[j], cv)
        pltpu.async_copy(p_hbm.at[rv], gpb, sa)
        pltpu.async_copy(q_hbm.at[cv], gqb, sb)

    def process(j, rv, cv, gpb, gqb, sa, sb):
        pltpu.make_async_copy(p_hbm.at[rv], gpb, sa).wait()
        pltpu.make_async_copy(q_hbm.at[cv], gqb, sb).wait()
        for v in range(CH // 16):
            sl = pl.ds(v * 16, 16)
            ir = rv[sl]
            ic = cv[sl]
            rx = plsc.load_gather(cxv, [ir]) - plsc.load_gather(cxv, [ic])
            ry = plsc.load_gather(cyv, [ir]) - plsc.load_gather(cyv, [ic])
            rz = plsc.load_gather(czv, [ir]) - plsc.load_gather(czv, [ic])
            d = jnp.clip(rx * rx + ry * ry + rz * rz, 1e-6, 1e6)
            rxb[sl] = rx
            ryb[sl] = ry
            rzb[sl] = rz
            db[sl] = d

        def addrow(r, _):
            for u in range(H // 16):
                su = pl.ds(u * 16, 16)
                gpb[r, su] = gpb[r, su] + gqb[r, su]
            return _
        lax.fori_loop(0, CH, addrow, None)
        eb = j * CH
        pltpu.sync_copy(gpb, gp_hbm.at[pl.ds(eb, CH)])
        pltpu.sync_copy(rxb, relt_hbm.at[0, pl.ds(eb, CH)])
        pltpu.sync_copy(ryb, relt_hbm.at[1, pl.ds(eb, CH)])
        pltpu.sync_copy(rzb, relt_hbm.at[2, pl.ds(eb, CH)])
        pltpu.sync_copy(db, relt_hbm.at[3, pl.ds(eb, CH)])

    fire(base, rv0, cv0, gpb0, gqb0, sa0, sb0)
    KH = KPW // 2

    def body(k, _):
        j0 = base + 2 * k
        fire(j0 + 1, rv1, cv1, gpb1, gqb1, sa1, sb1)
        process(j0, rv0, cv0, gpb0, gqb0, sa0, sb0)

        @pl.when(k < KH - 1)
        def _f():
            fire(j0 + 2, rv0, cv0, gpb0, gqb0, sa0, sb0)

        process(j0 + 1, rv1, cv1, gpb1, gqb1, sa1, sb1)
        return _

    lax.fori_loop(0, KH, body, None)


@functools.cache
def _sc_gather():
    return pl.kernel(
        _sc_gather_body,
        out_type=[
            jax.ShapeDtypeStruct((EP, H), jnp.float32),
            jax.ShapeDtypeStruct((4, EP), jnp.float32),
        ],
        mesh=_mesh(),
        scratch_types=[
            pltpu.VMEM((CH,), jnp.int32),
            pltpu.VMEM((CH,), jnp.int32),
            pltpu.VMEM((CH,), jnp.int32),
            pltpu.VMEM((CH,), jnp.int32),
            pltpu.VMEM((CH, H), jnp.float32),
            pltpu.VMEM((CH, H), jnp.float32),
            pltpu.VMEM((CH, H), jnp.float32),
            pltpu.VMEM((CH, H), jnp.float32),
            pltpu.VMEM((CH,), jnp.float32),
            pltpu.VMEM((CH,), jnp.float32),
            pltpu.VMEM((CH,), jnp.float32),
            pltpu.VMEM((CH,), jnp.float32),
            pltpu.VMEM((NPAD,), jnp.float32),
            pltpu.VMEM((NPAD,), jnp.float32),
            pltpu.VMEM((NPAD,), jnp.float32),
            pltpu.SemaphoreType.DMA,
            pltpu.SemaphoreType.DMA,
            pltpu.SemaphoreType.DMA,
            pltpu.SemaphoreType.DMA,
        ],
        compiler_params=pltpu.CompilerParams(needs_layout_passes=False),
    )


def _sc_scatter_body(m_hbm, s_hbm, ridx_hbm, zero_hbm,
                     agg_hbm, cu_hbm,
                     rv0, rv1, mb0, mb1, acc_sh, sm0, sm1):
    cid = lax.axis_index("c")
    sid = lax.axis_index("s")
    rows = pl.ds(sid * ROWS_PER_TILE, ROWS_PER_TILE)
    pltpu.sync_copy(zero_hbm.at[rows], acc_sh.at[rows])
    plsc.subcore_barrier()
    base = sid * KPT
    KH = KPT // 2

    def run(src_hbm):
        def fire(j, rv, mb, sm):
            pltpu.sync_copy(ridx_hbm.at[j], rv)
            pltpu.async_copy(src_hbm.at[pl.ds(j * CH, CH)], mb, sm)

        def proc(rv, mb, sm):
            pltpu.make_async_copy(src_hbm.at[pl.ds(0, CH)], mb, sm).wait()
            pltpu.sync_copy(mb, acc_sh.at[rv], add=True)

        fire(base, rv0, mb0, sm0)

        def body(k, carry):
            j0 = base + 2 * k
            fire(j0 + 1, rv1, mb1, sm1)
            proc(rv0, mb0, sm0)

            @pl.when(k < KH - 1)
            def _f():
                fire(j0 + 2, rv0, mb0, sm0)

            proc(rv1, mb1, sm1)
            return carry

        lax.fori_loop(0, KH, body, None)

    @pl.when(cid == 0)
    def _():
        run(m_hbm)

    @pl.when(cid == 1)
    def _():
        run(s_hbm)

    plsc.subcore_barrier()

    @pl.when(cid == 0)
    def _():
        pltpu.sync_copy(acc_sh.at[rows], agg_hbm.at[rows])

    @pl.when(cid == 1)
    def _():
        pltpu.sync_copy(acc_sh.at[rows], cu_hbm.at[rows])


@functools.cache
def _sc_scatter():
    return pl.kernel(
        _sc_scatter_body,
        out_type=[
            jax.ShapeDtypeStruct((NPAD, H), jnp.float32),
            jax.ShapeDtypeStruct((NPAD, H), jnp.float32),
        ],
        mesh=_mesh(),
        scratch_types=[
            pltpu.VMEM((CH,), jnp.int32),
            pltpu.VMEM((CH,), jnp.int32),
            pltpu.VMEM((CH, H), jnp.float32),
            pltpu.VMEM((CH, H), jnp.float32),
            pltpu.VMEM_SHARED((NPAD, H), jnp.float32),
            pltpu.SemaphoreType.DMA,
            pltpu.SemaphoreType.DMA,
        ],
    )


def kernel(z, atom_types, edge_index, Wlp, blp, Wap, bap, init_coords,
           eW1, eb1, eW2, eb2, nW1, nb1, nW2, nb2, cW1, cb1, cW2,
           gamma, beta, hW1, hb1, hW2, hb2):
    f32 = jnp.float32
    row = edge_index[0].astype(jnp.int32)
    col = edge_index[1].astype(jnp.int32)
    ridx = jnp.concatenate(
        [row, jnp.full((EP - E,), PAD_NODE, jnp.int32)]).reshape(CHUNKS_PAD, CH)
    cidx = jnp.concatenate(
        [col, jnp.zeros((EP - E,), jnp.int32)]).reshape(CHUNKS_PAD, CH)
    ridx_g = jnp.where(ridx >= N, 0, ridx)

    zr = jnp.broadcast_to(z[:, None, :], (B, A, L)).reshape(N, L)
    atp = jnp.pad(atom_types.astype(f32), ((0, 0), (0, 6)))
    wapp = jnp.pad(Wap.astype(f32), ((0, 6), (0, 0)))
    c16 = jnp.pad(
        jnp.broadcast_to(init_coords[None], (B, A, 3)).reshape(N, 3),
        ((0, 0), (0, 13)))

    blp2 = blp.reshape(1, H)
    bap2 = bap.reshape(1, H)
    sel = jnp.zeros((4, H), f32).at[0, 0].set(1.0).at[1, 1].set(1.0).at[2, 2].set(1.0)
    lane3 = jnp.zeros((1, H), f32).at[0, 3].set(1.0)
    zeros_big = jnp.zeros((NPAD, H), f32)
    hw2p = jnp.pad(hW2, ((0, 0), (0, 13)))
    hb2p = jnp.pad(hb2.reshape(1, 3), ((0, 0), (0, 13)))

    nf, P, Q = _precompute(zr, atp, Wlp, blp2, wapp, bap2,
                           eW1[0, :H], eW1[0, H:2 * H], eb1[0].reshape(1, H))

    for i in range(NL):
        c16p = jnp.pad(c16, ((0, NPAD - N), (0, 0)))
        cx, cy, cz = c16p[:, 0], c16p[:, 1], c16p[:, 2]
        g, relt = _sc_gather()(P, Q, cx, cy, cz, ridx_g, cidx)
        m, s128 = _edge_mlp(g, relt,
                            eW1[i, 2 * H].reshape(1, H), eW2[i],
                            eb2[i].reshape(1, H), cW1[i],
                            cb1[i].reshape(1, H), cW2[i].reshape(1, H),
                            sel, lane3)
        agg, cu128 = _sc_scatter()(m, s128, ridx, zeros_big)
        aggN = agg[:N]
        cu16 = cu128[:N, :16]
        if i < NL - 1:
            nf, c16, P, Q = _node_mid(
                nf, aggN, cu16, c16,
                nW1[i, :H], nW1[i, H:], nb1[i].reshape(1, H),
                nW2[i], nb2[i].reshape(1, H),
                gamma[i].reshape(1, H), beta[i].reshape(1, H),
                eW1[i + 1, :H], eW1[i + 1, H:2 * H], eb1[i + 1].reshape(1, H))
        else:
            out16 = _node_final(
                nf, aggN, cu16, c16,
                nW1[i, :H], nW1[i, H:], nb1[i].reshape(1, H),
                nW2[i], nb2[i].reshape(1, H),
                gamma[i].reshape(1, H), beta[i].reshape(1, H),
                hW1, hb1.reshape(1, H // 2), hw2p, hb2p)
    return out16[:, :3]


# speedup vs baseline: 3.7857x; 1.0066x over previous
import functools

import jax
import jax.numpy as jnp
from jax import lax
from jax.experimental import pallas as pl
from jax.experimental.pallas import tpu as pltpu
from jax.experimental.pallas import tpu_sc as plsc

B, A, H, L, F, E = 200, 50, 128, 128, 10, 320000
N = B * A
NL = 2

NC, NS = 2, 16
NW = NC * NS
CH = 128
CHUNKS = -(-E // CH)
KPW = 2 * (-(-(-(-CHUNKS // NW)) // 2))
CHUNKS_PAD = NW * KPW
KPT = CHUNKS_PAD // NS
EP = CHUNKS_PAD * CH
NPAD = 10240
ROWS_PER_TILE = NPAD // NS
PAD_NODE = N + 16

BE = 5120
GRID_E = EP // BE


def _precompute_body(zr_ref, at_ref, wlp_ref, blp_ref, wap_ref, bap_ref,
                     w1a_ref, w1b_ref, eb1_ref,
                     nf_ref, p_ref, q_ref):
    nf = (jnp.dot(zr_ref[...], wlp_ref[...], preferred_element_type=jnp.float32)
          + blp_ref[...]
          + jnp.dot(at_ref[...], wap_ref[...], preferred_element_type=jnp.float32)
          + bap_ref[...])
    nf_ref[...] = nf
    p_ref[...] = jnp.dot(nf, w1a_ref[...], preferred_element_type=jnp.float32) + eb1_ref[...]
    q_ref[...] = jnp.dot(nf, w1b_ref[...], preferred_element_type=jnp.float32)


def _precompute(zr, atp, wlp, blp, wapp, bap, w1a, w1b, eb1):
    return pl.pallas_call(
        _precompute_body,
        out_shape=[jax.ShapeDtypeStruct((N, H), jnp.float32)] * 3,
    )(zr, atp, wlp, blp, wapp, bap, w1a, w1b, eb1)


def _edge_body(gp_ref, rel_ref, wd_ref, ew2_ref, eb2_ref,
               cw1_ref, cb1_ref, cw2_ref, sel_ref, lane3_ref,
               m_ref, s_ref):
    rel = jnp.transpose(rel_ref[...], (1, 0))
    dist = rel[:, 3:4]
    g = gp_ref[...] + dist * wd_ref[...]
    m1 = jax.nn.silu(g)
    m = jax.nn.silu(
        jnp.dot(m1, ew2_ref[...], preferred_element_type=jnp.float32)
        + eb2_ref[...])
    cwp = jax.nn.silu(
        jnp.dot(m, cw1_ref[...], preferred_element_type=jnp.float32)
        + cb1_ref[...])
    cw = jnp.clip(jnp.sum(cwp * cw2_ref[...], axis=-1, keepdims=True), -1.0, 1.0)
    m_ref[...] = m
    s_ref[...] = cw * jnp.dot(rel, sel_ref[...],
                              preferred_element_type=jnp.float32) + lane3_ref[...]


def _edge_mlp(gp, relt, wd, ew2, eb2, cw1, cb1, cw2r, sel, lane3):
    full128 = pl.BlockSpec((H, H), lambda i: (0, 0))
    row128 = pl.BlockSpec((1, H), lambda i: (0, 0))
    return pl.pallas_call(
        _edge_body,
        grid=(GRID_E,),
        in_specs=[
            pl.BlockSpec((BE, H), lambda i: (i, 0)),
            pl.BlockSpec((4, BE), lambda i: (0, i)),
            row128, full128, row128, full128, row128, row128,
            pl.BlockSpec((4, H), lambda i: (0, 0)), row128,
        ],
        out_specs=[
            pl.BlockSpec((BE, H), lambda i: (i, 0)),
            pl.BlockSpec((BE, H), lambda i: (i, 0)),
        ],
        out_shape=[
            jax.ShapeDtypeStruct((EP, H), jnp.float32),
            jax.ShapeDtypeStruct((EP, H), jnp.float32),
        ],
    )(gp, relt, wd, ew2, eb2, cw1, cb1, cw2r, sel, lane3)


def _layernorm(xn, gamma, beta):
    mu = jnp.mean(xn, axis=-1, keepdims=True)
    var = jnp.mean((xn - mu) ** 2, axis=-1, keepdims=True)
    return (xn - mu) * jax.lax.rsqrt(var + 1e-5) * gamma + beta


def _node_mid_body(nf_ref, agg_ref, cu_ref, c_ref,
                   nw1a_ref, nw1b_ref, nb1_ref, nw2_ref, nb2_ref,
                   gam_ref, bet_ref, w1a_ref, w1b_ref, eb1_ref,
                   nf2_ref, c2_ref, p_ref, q_ref):
    nf = nf_ref[...]
    agg = agg_ref[...]
    xn = jax.nn.silu(
        jnp.dot(nf, nw1a_ref[...], preferred_element_type=jnp.float32)
        + jnp.dot(agg, nw1b_ref[...], preferred_element_type=jnp.float32)
        + nb1_ref[...])
    xn = jnp.dot(xn, nw2_ref[...], preferred_element_type=jnp.float32) + nb2_ref[...]
    xn = _layernorm(xn, gam_ref[...], bet_ref[...])
    nf2_ref[...] = xn
    cu = cu_ref[...]
    c2_ref[...] = c_ref[...] + cu / (cu[:, 3:4] + 1e-6)
    p_ref[...] = jnp.dot(xn, w1a_ref[...], preferred_element_type=jnp.float32) + eb1_ref[...]
    q_ref[...] = jnp.dot(xn, w1b_ref[...], preferred_element_type=jnp.float32)


def _node_mid(nf, agg, cu16, c16, nw1a, nw1b, nb1, nw2, nb2,
              gam, bet, w1a, w1b, eb1):
    return pl.pallas_call(
        _node_mid_body,
        out_shape=[
            jax.ShapeDtypeStruct((N, H), jnp.float32),
            jax.ShapeDtypeStruct((N, 16), jnp.float32),
            jax.ShapeDtypeStruct((N, H), jnp.float32),
            jax.ShapeDtypeStruct((N, H), jnp.float32),
        ],
    )(nf, agg, cu16, c16, nw1a, nw1b, nb1, nw2, nb2, gam, bet, w1a, w1b, eb1)


def _node_final_body(nf_ref, agg_ref, cu_ref, c_ref,
                     nw1a_ref, nw1b_ref, nb1_ref, nw2_ref, nb2_ref,
                     gam_ref, bet_ref, hw1_ref, hb1_ref, hw2_ref, hb2_ref,
                     out_ref):
    nf = nf_ref[...]
    agg = agg_ref[...]
    xn = jax.nn.silu(
        jnp.dot(nf, nw1a_ref[...], preferred_element_type=jnp.float32)
        + jnp.dot(agg, nw1b_ref[...], preferred_element_type=jnp.float32)
        + nb1_ref[...])
    xn = jnp.dot(xn, nw2_ref[...], preferred_element_type=jnp.float32) + nb2_ref[...]
    xn = _layernorm(xn, gam_ref[...], bet_ref[...])
    cu = cu_ref[...]
    c2 = c_ref[...] + cu / (cu[:, 3:4] + 1e-6)
    hid = jax.nn.silu(
        jnp.dot(xn, hw1_ref[...], preferred_element_type=jnp.float32)
        + hb1_ref[...])
    cd = jnp.dot(hid, hw2_ref[...], preferred_element_type=jnp.float32) + hb2_ref[...]
    out_ref[...] = c2 + cd


def _node_final(nf, agg, cu16, c16, nw1a, nw1b, nb1, nw2, nb2,
                gam, bet, hw1, hb1, hw2p, hb2p):
    return pl.pallas_call(
        _node_final_body,
        out_shape=jax.ShapeDtypeStruct((N, 16), jnp.float32),
    )(nf, agg, cu16, c16, nw1a, nw1b, nb1, nw2, nb2, gam, bet,
      hw1, hb1, hw2p, hb2p)


@functools.cache
def _mesh():
    return plsc.VectorSubcoreMesh(core_axis_name="c", subcore_axis_name="s",
                                  num_cores=NC, num_subcores=NS)


def _sc_gather_body(p_hbm, q_hbm, cx_hbm, cy_hbm, cz_hbm, ridx_hbm, cidx_hbm,
                    gp_hbm, relt_hbm,
                    rv0, cv0, rv1, cv1, gpb0, gqb0, gpb1, gqb1,
                    rxb, ryb, rzb, db, cxv, cyv, czv,
                    sa0, sb0, sa1, sb1, sw0, sw1):
    wid = lax.axis_index("s") * NC + lax.axis_index("c")
    base = wid * KPW
    pltpu.sync_copy(cx_hbm, cxv)
    pltpu.sync_copy(cy_hbm, cyv)
    pltpu.sync_copy(cz_hbm, czv)

    def fire(j, rv, cv, gpb, gqb, sa, sb):
        pltpu.sync_copy(ridx_hbm.at[j], rv)
        pltpu.sync_copy(cidx_hbm.at[j], cv)
        pltpu.async_copy(p_hbm.at[rv], gpb, sa)
        pltpu.async_copy(q_hbm.at[cv], gqb, sb)

    def drain_write(gpb, sw):
        pltpu.make_async_copy(gpb, gp_hbm.at[pl.ds(0, CH)], sw).wait()

    def process(j, rv, cv, gpb, gqb, sa, sb, sw):
        pltpu.make_async_copy(p_hbm.at[rv], gpb, sa).wait()
        pltpu.make_async_copy(q_hbm.at[cv], gqb, sb).wait()
        for v in range(CH // 16):
            sl = pl.ds(v * 16, 16)
            ir = rv[sl]
            ic = cv[sl]
            rx = plsc.load_gather(cxv, [ir]) - plsc.load_gather(cxv, [ic])
            ry = plsc.load_gather(cyv, [ir]) - plsc.load_gather(cyv, [ic])
            rz = plsc.load_gather(czv, [ir]) - plsc.load_gather(czv, [ic])
            d = jnp.clip(rx * rx + ry * ry + rz * rz, 1e-6, 1e6)
            rxb[sl] = rx
            ryb[sl] = ry
            rzb[sl] = rz
            db[sl] = d

        def addrow(r, _):
            for u in range(H // 16):
                su = pl.ds(u * 16, 16)
                gpb[r, su] = gpb[r, su] + gqb[r, su]
            return _
        lax.fori_loop(0, CH, addrow, None)
        eb = j * CH
        pltpu.async_copy(gpb, gp_hbm.at[pl.ds(eb, CH)], sw)
        pltpu.sync_copy(rxb, relt_hbm.at[0, pl.ds(eb, CH)])
        pltpu.sync_copy(ryb, relt_hbm.at[1, pl.ds(eb, CH)])
        pltpu.sync_copy(rzb, relt_hbm.at[2, pl.ds(eb, CH)])
        pltpu.sync_copy(db, relt_hbm.at[3, pl.ds(eb, CH)])

    fire(base, rv0, cv0, gpb0, gqb0, sa0, sb0)
    fire(base + 1, rv1, cv1, gpb1, gqb1, sa1, sb1)
    KH = KPW // 2

    def body(k, _):
        j0 = base + 2 * k
        process(j0, rv0, cv0, gpb0, gqb0, sa0, sb0, sw0)

        @pl.when(k < KH - 1)
        def _f0():
            drain_write(gpb0, sw0)
            fire(j0 + 2, rv0, cv0, gpb0, gqb0, sa0, sb0)

        process(j0 + 1, rv1, cv1, gpb1, gqb1, sa1, sb1, sw1)

        @pl.when(k < KH - 1)
        def _f1():
            drain_write(gpb1, sw1)
            fire(j0 + 3, rv1, cv1, gpb1, gqb1, sa1, sb1)

        return _

    lax.fori_loop(0, KH, body, None)
    drain_write(gpb0, sw0)
    drain_write(gpb1, sw1)


@functools.cache
def _sc_gather():
    return pl.kernel(
        _sc_gather_body,
        out_type=[
            jax.ShapeDtypeStruct((EP, H), jnp.float32),
            jax.ShapeDtypeStruct((4, EP), jnp.float32),
        ],
        mesh=_mesh(),
        scratch_types=[
            pltpu.VMEM((CH,), jnp.int32),
            pltpu.VMEM((CH,), jnp.int32),
            pltpu.VMEM((CH,), jnp.int32),
            pltpu.VMEM((CH,), jnp.int32),
            pltpu.VMEM((CH, H), jnp.float32),
            pltpu.VMEM((CH, H), jnp.float32),
            pltpu.VMEM((CH, H), jnp.float32),
            pltpu.VMEM((CH, H), jnp.float32),
            pltpu.VMEM((CH,), jnp.float32),
            pltpu.VMEM((CH,), jnp.float32),
            pltpu.VMEM((CH,), jnp.float32),
            pltpu.VMEM((CH,), jnp.float32),
            pltpu.VMEM((NPAD,), jnp.float32),
            pltpu.VMEM((NPAD,), jnp.float32),
            pltpu.VMEM((NPAD,), jnp.float32),
            pltpu.SemaphoreType.DMA,
            pltpu.SemaphoreType.DMA,
            pltpu.SemaphoreType.DMA,
            pltpu.SemaphoreType.DMA,
            pltpu.SemaphoreType.DMA,
            pltpu.SemaphoreType.DMA,
        ],
        compiler_params=pltpu.CompilerParams(needs_layout_passes=False),
    )


def _sc_scatter_body(m_hbm, s_hbm, ridx_hbm, zero_hbm,
                     agg_hbm, cu_hbm,
                     rv0, rv1, mb0, mb1, acc_sh, sm0, sm1):
    cid = lax.axis_index("c")
    sid = lax.axis_index("s")
    rows = pl.ds(sid * ROWS_PER_TILE, ROWS_PER_TILE)
    pltpu.sync_copy(zero_hbm.at[rows], acc_sh.at[rows])
    plsc.subcore_barrier()
    base = sid * KPT
    KH = KPT // 2

    def run(src_hbm):
        def fire(j, rv, mb, sm):
            pltpu.sync_copy(ridx_hbm.at[j], rv)
            pltpu.async_copy(src_hbm.at[pl.ds(j * CH, CH)], mb, sm)

        def proc(rv, mb, sm):
            pltpu.make_async_copy(src_hbm.at[pl.ds(0, CH)], mb, sm).wait()
            pltpu.sync_copy(mb, acc_sh.at[rv], add=True)

        fire(base, rv0, mb0, sm0)

        def body(k, carry):
            j0 = base + 2 * k
            fire(j0 + 1, rv1, mb1, sm1)
            proc(rv0, mb0, sm0)

            @pl.when(k < KH - 1)
            def _f():
                fire(j0 + 2, rv0, mb0, sm0)

            proc(rv1, mb1, sm1)
            return carry

        lax.fori_loop(0, KH, body, None)

    @pl.when(cid == 0)
    def _():
        run(m_hbm)

    @pl.when(cid == 1)
    def _():
        run(s_hbm)

    plsc.subcore_barrier()

    @pl.when(cid == 0)
    def _():
        pltpu.sync_copy(acc_sh.at[rows], agg_hbm.at[rows])

    @pl.when(cid == 1)
    def _():
        pltpu.sync_copy(acc_sh.at[rows], cu_hbm.at[rows])


@functools.cache
def _sc_scatter():
    return pl.kernel(
        _sc_scatter_body,
        out_type=[
            jax.ShapeDtypeStruct((NPAD, H), jnp.float32),
            jax.ShapeDtypeStruct((NPAD, H), jnp.float32),
        ],
        mesh=_mesh(),
        scratch_types=[
            pltpu.VMEM((CH,), jnp.int32),
            pltpu.VMEM((CH,), jnp.int32),
            pltpu.VMEM((CH, H), jnp.float32),
            pltpu.VMEM((CH, H), jnp.float32),
            pltpu.VMEM_SHARED((NPAD, H), jnp.float32),
            pltpu.SemaphoreType.DMA,
            pltpu.SemaphoreType.DMA,
        ],
    )


def kernel(z, atom_types, edge_index, Wlp, blp, Wap, bap, init_coords,
           eW1, eb1, eW2, eb2, nW1, nb1, nW2, nb2, cW1, cb1, cW2,
           gamma, beta, hW1, hb1, hW2, hb2):
    f32 = jnp.float32
    row = edge_index[0].astype(jnp.int32)
    col = edge_index[1].astype(jnp.int32)
    ridx = jnp.concatenate(
        [row, jnp.full((EP - E,), PAD_NODE, jnp.int32)]).reshape(CHUNKS_PAD, CH)
    cidx = jnp.concatenate(
        [col, jnp.zeros((EP - E,), jnp.int32)]).reshape(CHUNKS_PAD, CH)
    ridx_g = jnp.where(ridx >= N, 0, ridx)

    zr = jnp.broadcast_to(z[:, None, :], (B, A, L)).reshape(N, L)
    atp = jnp.pad(atom_types.astype(f32), ((0, 0), (0, 6)))
    wapp = jnp.pad(Wap.astype(f32), ((0, 6), (0, 0)))
    c16 = jnp.pad(
        jnp.broadcast_to(init_coords[None], (B, A, 3)).reshape(N, 3),
        ((0, 0), (0, 13)))

    blp2 = blp.reshape(1, H)
    bap2 = bap.reshape(1, H)
    sel = jnp.zeros((4, H), f32).at[0, 0].set(1.0).at[1, 1].set(1.0).at[2, 2].set(1.0)
    lane3 = jnp.zeros((1, H), f32).at[0, 3].set(1.0)
    zeros_big = jnp.zeros((NPAD, H), f32)
    hw2p = jnp.pad(hW2, ((0, 0), (0, 13)))
    hb2p = jnp.pad(hb2.reshape(1, 3), ((0, 0), (0, 13)))

    nf, P, Q = _precompute(zr, atp, Wlp, blp2, wapp, bap2,
                           eW1[0, :H], eW1[0, H:2 * H], eb1[0].reshape(1, H))

    for i in range(NL):
        c16p = jnp.pad(c16, ((0, NPAD - N), (0, 0)))
        cx, cy, cz = c16p[:, 0], c16p[:, 1], c16p[:, 2]
        g, relt = _sc_gather()(P, Q, cx, cy, cz, ridx_g, cidx)
        m, s128 = _edge_mlp(g, relt,
                            eW1[i, 2 * H].reshape(1, H), eW2[i],
                            eb2[i].reshape(1, H), cW1[i],
                            cb1[i].reshape(1, H), cW2[i].reshape(1, H),
                            sel, lane3)
        agg, cu128 = _sc_scatter()(m, s128, ridx, zeros_big)
        aggN = agg[:N]
        cu16 = cu128[:N, :16]
        if i < NL - 1:
            nf, c16, P, Q = _node_mid(
                nf, aggN, cu16, c16,
                nW1[i, :H], nW1[i, H:], nb1[i].reshape(1, H),
                nW2[i], nb2[i].reshape(1, H),
                gamma[i].reshape(1, H), beta[i].reshape(1, H),
                eW1[i + 1, :H], eW1[i + 1, H:2 * H], eb1[i + 1].reshape(1, H))
        else:
            out16 = _node_final(
                nf, aggN, cu16, c16,
                nW1[i, :H], nW1[i, H:], nb1[i].reshape(1, H),
                nW2[i], nb2[i].reshape(1, H),
                gamma[i].reshape(1, H), beta[i].reshape(1, H),
                hW1, hb1.reshape(1, H // 2), hw2p, hb2p)
    return out16[:, :3]


# speedup vs baseline: 3.7877x; 1.0005x over previous
import functools

import jax
import jax.numpy as jnp
from jax import lax
from jax.experimental import pallas as pl
from jax.experimental.pallas import tpu as pltpu
from jax.experimental.pallas import tpu_sc as plsc

B, A, H, L, F, E = 200, 50, 128, 128, 10, 320000
N = B * A
NL = 2

NC, NS = 2, 16
NW = NC * NS
CH = 128
CHUNKS = -(-E // CH)
KPW = 2 * (-(-(-(-CHUNKS // NW)) // 2))
CHUNKS_PAD = NW * KPW
KPT = CHUNKS_PAD // NS
EP = CHUNKS_PAD * CH
NPAD = 10240
ROWS_PER_TILE = NPAD // NS
PAD_NODE = N + 16

BE = 5120
GRID_E = EP // BE


def _precompute_body(zr_ref, at_ref, wlp_ref, blp_ref, wap_ref, bap_ref,
                     w1a_ref, w1b_ref, eb1_ref,
                     nf_ref, p_ref, q_ref):
    nf = (jnp.dot(zr_ref[...], wlp_ref[...], preferred_element_type=jnp.float32)
          + blp_ref[...]
          + jnp.dot(at_ref[...], wap_ref[...], preferred_element_type=jnp.float32)
          + bap_ref[...])
    nf_ref[...] = nf
    p_ref[...] = jnp.dot(nf, w1a_ref[...], preferred_element_type=jnp.float32) + eb1_ref[...]
    q_ref[...] = jnp.dot(nf, w1b_ref[...], preferred_element_type=jnp.float32)


def _precompute(zr, atp, wlp, blp, wapp, bap, w1a, w1b, eb1):
    return pl.pallas_call(
        _precompute_body,
        out_shape=[jax.ShapeDtypeStruct((N, H), jnp.float32)] * 3,
    )(zr, atp, wlp, blp, wapp, bap, w1a, w1b, eb1)


def _edge_body(gp_ref, rel_ref, wd_ref, ew2_ref, eb2_ref,
               cw1_ref, cb1_ref, cw2_ref, sel_ref, lane3_ref,
               m_ref, s_ref):
    rel = jnp.transpose(rel_ref[...], (1, 0))
    dist = rel[:, 3:4]
    g = gp_ref[...] + dist * wd_ref[...]
    m1 = jax.nn.silu(g)
    m = jax.nn.silu(
        jnp.dot(m1, ew2_ref[...], preferred_element_type=jnp.float32)
        + eb2_ref[...])
    cwp = jax.nn.silu(
        jnp.dot(m, cw1_ref[...], preferred_element_type=jnp.float32)
        + cb1_ref[...])
    cw = jnp.clip(jnp.sum(cwp * cw2_ref[...], axis=-1, keepdims=True), -1.0, 1.0)
    m_ref[...] = m
    s_ref[...] = cw * jnp.dot(rel, sel_ref[...],
                              preferred_element_type=jnp.float32) + lane3_ref[...]


def _edge_mlp(gp, relt, wd, ew2, eb2, cw1, cb1, cw2r, sel, lane3):
    full128 = pl.BlockSpec((H, H), lambda i: (0, 0))
    row128 = pl.BlockSpec((1, H), lambda i: (0, 0))
    return pl.pallas_call(
        _edge_body,
        grid=(GRID_E,),
        in_specs=[
            pl.BlockSpec((BE, H), lambda i: (i, 0)),
            pl.BlockSpec((4, BE), lambda i: (0, i)),
            row128, full128, row128, full128, row128, row128,
            pl.BlockSpec((4, H), lambda i: (0, 0)), row128,
        ],
        out_specs=[
            pl.BlockSpec((BE, H), lambda i: (i, 0)),
            pl.BlockSpec((BE, H), lambda i: (i, 0)),
        ],
        out_shape=[
            jax.ShapeDtypeStruct((EP, H), jnp.float32),
            jax.ShapeDtypeStruct((EP, H), jnp.float32),
        ],
    )(gp, relt, wd, ew2, eb2, cw1, cb1, cw2r, sel, lane3)


def _layernorm(xn, gamma, beta):
    mu = jnp.mean(xn, axis=-1, keepdims=True)
    var = jnp.mean((xn - mu) ** 2, axis=-1, keepdims=True)
    return (xn - mu) * jax.lax.rsqrt(var + 1e-5) * gamma + beta


def _node_mid_body(nf_ref, agg_ref, cu_ref, c_ref,
                   nw1a_ref, nw1b_ref, nb1_ref, nw2_ref, nb2_ref,
                   gam_ref, bet_ref, w1a_ref, w1b_ref, eb1_ref,
                   nf2_ref, c2_ref, p_ref, q_ref):
    nf = nf_ref[...]
    agg = agg_ref[...]
    xn = jax.nn.silu(
        jnp.dot(nf, nw1a_ref[...], preferred_element_type=jnp.float32)
        + jnp.dot(agg, nw1b_ref[...], preferred_element_type=jnp.float32)
        + nb1_ref[...])
    xn = jnp.dot(xn, nw2_ref[...], preferred_element_type=jnp.float32) + nb2_ref[...]
    xn = _layernorm(xn, gam_ref[...], bet_ref[...])
    nf2_ref[...] = xn
    cu = cu_ref[...]
    c2_ref[...] = c_ref[...] + cu / (cu[:, 3:4] + 1e-6)
    p_ref[...] = jnp.dot(xn, w1a_ref[...], preferred_element_type=jnp.float32) + eb1_ref[...]
    q_ref[...] = jnp.dot(xn, w1b_ref[...], preferred_element_type=jnp.float32)


def _node_mid(nf, agg, cu16, c16, nw1a, nw1b, nb1, nw2, nb2,
              gam, bet, w1a, w1b, eb1):
    return pl.pallas_call(
        _node_mid_body,
        out_shape=[
            jax.ShapeDtypeStruct((N, H), jnp.float32),
            jax.ShapeDtypeStruct((N, 16), jnp.float32),
            jax.ShapeDtypeStruct((N, H), jnp.float32),
            jax.ShapeDtypeStruct((N, H), jnp.float32),
        ],
    )(nf, agg, cu16, c16, nw1a, nw1b, nb1, nw2, nb2, gam, bet, w1a, w1b, eb1)


def _node_final_body(nf_ref, agg_ref, cu_ref, c_ref,
                     nw1a_ref, nw1b_ref, nb1_ref, nw2_ref, nb2_ref,
                     gam_ref, bet_ref, hw1_ref, hb1_ref, hw2_ref, hb2_ref,
                     out_ref):
    nf = nf_ref[...]
    agg = agg_ref[...]
    xn = jax.nn.silu(
        jnp.dot(nf, nw1a_ref[...], preferred_element_type=jnp.float32)
        + jnp.dot(agg, nw1b_ref[...], preferred_element_type=jnp.float32)
        + nb1_ref[...])
    xn = jnp.dot(xn, nw2_ref[...], preferred_element_type=jnp.float32) + nb2_ref[...]
    xn = _layernorm(xn, gam_ref[...], bet_ref[...])
    cu = cu_ref[...]
    c2 = c_ref[...] + cu / (cu[:, 3:4] + 1e-6)
    hid = jax.nn.silu(
        jnp.dot(xn, hw1_ref[...], preferred_element_type=jnp.float32)
        + hb1_ref[...])
    cd = jnp.dot(hid, hw2_ref[...], preferred_element_type=jnp.float32) + hb2_ref[...]
    out_ref[...] = c2 + cd


def _node_final(nf, agg, cu16, c16, nw1a, nw1b, nb1, nw2, nb2,
                gam, bet, hw1, hb1, hw2p, hb2p):
    return pl.pallas_call(
        _node_final_body,
        out_shape=jax.ShapeDtypeStruct((N, 16), jnp.float32),
    )(nf, agg, cu16, c16, nw1a, nw1b, nb1, nw2, nb2, gam, bet,
      hw1, hb1, hw2p, hb2p)


@functools.cache
def _mesh():
    return plsc.VectorSubcoreMesh(core_axis_name="c", subcore_axis_name="s",
                                  num_cores=NC, num_subcores=NS)


def _sc_gather_body(p_hbm, q_hbm, cx_hbm, cy_hbm, cz_hbm, ridx_hbm, cidx_hbm,
                    gp_hbm, relt_hbm,
                    rv0, cv0, rv1, cv1, gpb0, gqb0, gpb1, gqb1,
                    rxb, ryb, rzb, db, cxv, cyv, czv,
                    sa0, sb0, sa1, sb1, sw0, sw1):
    wid = lax.axis_index("s") * NC + lax.axis_index("c")
    base = wid * KPW
    pltpu.sync_copy(cx_hbm, cxv)
    pltpu.sync_copy(cy_hbm, cyv)
    pltpu.sync_copy(cz_hbm, czv)

    def fire(j, rv, cv, gpb, gqb, sa, sb):
        pltpu.sync_copy(ridx_hbm.at[j], rv)
        pltpu.sync_copy(cidx_hbm.at[j], cv)
        pltpu.async_copy(p_hbm.at[rv], gpb, sa)
        pltpu.async_copy(q_hbm.at[cv], gqb, sb)

    def drain_write(gpb, sw):
        pltpu.make_async_copy(gpb, gp_hbm.at[pl.ds(0, CH)], sw).wait()

    def process(j, rv, cv, gpb, gqb, sa, sb, sw):
        pltpu.make_async_copy(p_hbm.at[rv], gpb, sa).wait()
        pltpu.make_async_copy(q_hbm.at[cv], gqb, sb).wait()
        for v in range(CH // 16):
            sl = pl.ds(v * 16, 16)
            ir = rv[sl]
            ic = cv[sl]
            rx = plsc.load_gather(cxv, [ir]) - plsc.load_gather(cxv, [ic])
            ry = plsc.load_gather(cyv, [ir]) - plsc.load_gather(cyv, [ic])
            rz = plsc.load_gather(czv, [ir]) - plsc.load_gather(czv, [ic])
            d = jnp.clip(rx * rx + ry * ry + rz * rz, 1e-6, 1e6)
            rxb[sl] = rx
            ryb[sl] = ry
            rzb[sl] = rz
            db[sl] = d

        def addrow(r, _):
            for u in range(H // 16):
                su = pl.ds(u * 16, 16)
                gpb[r, su] = gpb[r, su] + gqb[r, su]
            return _
        lax.fori_loop(0, CH, addrow, None)
        eb = j * CH
        pltpu.async_copy(gpb, gp_hbm.at[pl.ds(eb, CH)], sw)
        pltpu.sync_copy(rxb, relt_hbm.at[0, pl.ds(eb, CH)])
        pltpu.sync_copy(ryb, relt_hbm.at[1, pl.ds(eb, CH)])
        pltpu.sync_copy(rzb, relt_hbm.at[2, pl.ds(eb, CH)])
        pltpu.sync_copy(db, relt_hbm.at[3, pl.ds(eb, CH)])

    fire(base, rv0, cv0, gpb0, gqb0, sa0, sb0)
    fire(base + 1, rv1, cv1, gpb1, gqb1, sa1, sb1)
    KH = KPW // 2

    def body(k, _):
        j0 = base + 2 * k
        process(j0, rv0, cv0, gpb0, gqb0, sa0, sb0, sw0)

        @pl.when(k < KH - 1)
        def _f0():
            drain_write(gpb0, sw0)
            fire(j0 + 2, rv0, cv0, gpb0, gqb0, sa0, sb0)

        process(j0 + 1, rv1, cv1, gpb1, gqb1, sa1, sb1, sw1)

        @pl.when(k < KH - 1)
        def _f1():
            drain_write(gpb1, sw1)
            fire(j0 + 3, rv1, cv1, gpb1, gqb1, sa1, sb1)

        return _

    lax.fori_loop(0, KH, body, None)
    drain_write(gpb0, sw0)
    drain_write(gpb1, sw1)


@functools.cache
def _sc_gather():
    return pl.kernel(
        _sc_gather_body,
        out_type=[
            jax.ShapeDtypeStruct((EP, H), jnp.float32),
            jax.ShapeDtypeStruct((4, EP), jnp.float32),
        ],
        mesh=_mesh(),
        scratch_types=[
            pltpu.VMEM((CH,), jnp.int32),
            pltpu.VMEM((CH,), jnp.int32),
            pltpu.VMEM((CH,), jnp.int32),
            pltpu.VMEM((CH,), jnp.int32),
            pltpu.VMEM((CH, H), jnp.float32),
            pltpu.VMEM((CH, H), jnp.float32),
            pltpu.VMEM((CH, H), jnp.float32),
            pltpu.VMEM((CH, H), jnp.float32),
            pltpu.VMEM((CH,), jnp.float32),
            pltpu.VMEM((CH,), jnp.float32),
            pltpu.VMEM((CH,), jnp.float32),
            pltpu.VMEM((CH,), jnp.float32),
            pltpu.VMEM((NPAD,), jnp.float32),
            pltpu.VMEM((NPAD,), jnp.float32),
            pltpu.VMEM((NPAD,), jnp.float32),
            pltpu.SemaphoreType.DMA,
            pltpu.SemaphoreType.DMA,
            pltpu.SemaphoreType.DMA,
            pltpu.SemaphoreType.DMA,
            pltpu.SemaphoreType.DMA,
            pltpu.SemaphoreType.DMA,
        ],
        compiler_params=pltpu.CompilerParams(needs_layout_passes=False),
    )


def _sc_scatter_body(m_hbm, s_hbm, ridx_hbm, zero_hbm,
                     agg_hbm, cu_hbm,
                     rv0, rv1, mb0, mb1, acc_sh, sm0, sm1, ss0, ss1):
    cid = lax.axis_index("c")
    sid = lax.axis_index("s")
    rows = pl.ds(sid * ROWS_PER_TILE, ROWS_PER_TILE)
    pltpu.sync_copy(zero_hbm.at[rows], acc_sh.at[rows])
    plsc.subcore_barrier()
    base = sid * KPT
    KH = KPT // 2

    def run(src_hbm):
        def fire(j, rv, mb, sm):
            pltpu.sync_copy(ridx_hbm.at[j], rv)
            pltpu.async_copy(src_hbm.at[pl.ds(j * CH, CH)], mb, sm)

        def proc(rv, mb, sm, ss):
            pltpu.make_async_copy(src_hbm.at[pl.ds(0, CH)], mb, sm).wait()
            pltpu.async_copy(mb, acc_sh.at[rv], ss, add=True)

        def drain_add(rv, mb, ss):
            pltpu.make_async_copy(mb, acc_sh.at[rv], ss).wait()

        fire(base, rv0, mb0, sm0)
        fire(base + 1, rv1, mb1, sm1)

        def body(k, carry):
            j0 = base + 2 * k
            proc(rv0, mb0, sm0, ss0)

            @pl.when(k < KH - 1)
            def _f0():
                drain_add(rv0, mb0, ss0)
                fire(j0 + 2, rv0, mb0, sm0)

            proc(rv1, mb1, sm1, ss1)

            @pl.when(k < KH - 1)
            def _f1():
                drain_add(rv1, mb1, ss1)
                fire(j0 + 3, rv1, mb1, sm1)

            return carry

        lax.fori_loop(0, KH, body, None)
        drain_add(rv0, mb0, ss0)
        drain_add(rv1, mb1, ss1)

    @pl.when(cid == 0)
    def _():
        run(m_hbm)

    @pl.when(cid == 1)
    def _():
        run(s_hbm)

    plsc.subcore_barrier()

    @pl.when(cid == 0)
    def _():
        pltpu.sync_copy(acc_sh.at[rows], agg_hbm.at[rows])

    @pl.when(cid == 1)
    def _():
        pltpu.sync_copy(acc_sh.at[rows], cu_hbm.at[rows])


@functools.cache
def _sc_scatter():
    return pl.kernel(
        _sc_scatter_body,
        out_type=[
            jax.ShapeDtypeStruct((NPAD, H), jnp.float32),
            jax.ShapeDtypeStruct((NPAD, H), jnp.float32),
        ],
        mesh=_mesh(),
        scratch_types=[
            pltpu.VMEM((CH,), jnp.int32),
            pltpu.VMEM((CH,), jnp.int32),
            pltpu.VMEM((CH, H), jnp.float32),
            pltpu.VMEM((CH, H), jnp.float32),
            pltpu.VMEM_SHARED((NPAD, H), jnp.float32),
            pltpu.SemaphoreType.DMA,
            pltpu.SemaphoreType.DMA,
            pltpu.SemaphoreType.DMA,
            pltpu.SemaphoreType.DMA,
        ],
    )


def kernel(z, atom_types, edge_index, Wlp, blp, Wap, bap, init_coords,
           eW1, eb1, eW2, eb2, nW1, nb1, nW2, nb2, cW1, cb1, cW2,
           gamma, beta, hW1, hb1, hW2, hb2):
    f32 = jnp.float32
    row = edge_index[0].astype(jnp.int32)
    col = edge_index[1].astype(jnp.int32)
    ridx = jnp.concatenate(
        [row, jnp.full((EP - E,), PAD_NODE, jnp.int32)]).reshape(CHUNKS_PAD, CH)
    cidx = jnp.concatenate(
        [col, jnp.zeros((EP - E,), jnp.int32)]).reshape(CHUNKS_PAD, CH)
    ridx_g = jnp.where(ridx >= N, 0, ridx)

    zr = jnp.broadcast_to(z[:, None, :], (B, A, L)).reshape(N, L)
    atp = jnp.pad(atom_types.astype(f32), ((0, 0), (0, 6)))
    wapp = jnp.pad(Wap.astype(f32), ((0, 6), (0, 0)))
    c16 = jnp.pad(
        jnp.broadcast_to(init_coords[None], (B, A, 3)).reshape(N, 3),
        ((0, 0), (0, 13)))

    blp2 = blp.reshape(1, H)
    bap2 = bap.reshape(1, H)
    sel = jnp.zeros((4, H), f32).at[0, 0].set(1.0).at[1, 1].set(1.0).at[2, 2].set(1.0)
    lane3 = jnp.zeros((1, H), f32).at[0, 3].set(1.0)
    zeros_big = jnp.zeros((NPAD, H), f32)
    hw2p = jnp.pad(hW2, ((0, 0), (0, 13)))
    hb2p = jnp.pad(hb2.reshape(1, 3), ((0, 0), (0, 13)))

    nf, P, Q = _precompute(zr, atp, Wlp, blp2, wapp, bap2,
                           eW1[0, :H], eW1[0, H:2 * H], eb1[0].reshape(1, H))

    for i in range(NL):
        c16p = jnp.pad(c16, ((0, NPAD - N), (0, 0)))
        cx, cy, cz = c16p[:, 0], c16p[:, 1], c16p[:, 2]
        g, relt = _sc_gather()(P, Q, cx, cy, cz, ridx_g, cidx)
        m, s128 = _edge_mlp(g, relt,
                            eW1[i, 2 * H].reshape(1, H), eW2[i],
                            eb2[i].reshape(1, H), cW1[i],
                            cb1[i].reshape(1, H), cW2[i].reshape(1, H),
                            sel, lane3)
        agg, cu128 = _sc_scatter()(m, s128, ridx, zeros_big)
        aggN = agg[:N]
        cu16 = cu128[:N, :16]
        if i < NL - 1:
            nf, c16, P, Q = _node_mid(
                nf, aggN, cu16, c16,
                nW1[i, :H], nW1[i, H:], nb1[i].reshape(1, H),
                nW2[i], nb2[i].reshape(1, H),
                gamma[i].reshape(1, H), beta[i].reshape(1, H),
                eW1[i + 1, :H], eW1[i + 1, H:2 * H], eb1[i + 1].reshape(1, H))
        else:
            out16 = _node_final(
                nf, aggN, cu16, c16,
                nW1[i, :H], nW1[i, H:], nb1[i].reshape(1, H),
                nW2[i], nb2[i].reshape(1, H),
                gamma[i].reshape(1, H), beta[i].reshape(1, H),
                hW1, hb1.reshape(1, H // 2), hw2p, hb2p)
    return out16[:, :3]
